# baseline (speedup 1.0000x reference)
import dataclasses
import functools

import jax
import jax.numpy as jnp
from jax import lax
from jax.experimental import pallas as pl
from jax.experimental.pallas import tpu as pltpu
from jax.experimental.pallas import tpu_sc as plsc

N = 10000
D = 512
BLK = 400

BROWS = 128
NBUCK = (N + BROWS - 1) // BROWS
NITEM = 2 * NBUCK
CHUNK = 48
E = 78000
DATA_CAP = 81696
CAP = DATA_CAP + 256
PER = 4880
NGRP = PER // 16
NHIST = 10128
SENT_NODE = 10008
SENT_DST = 10112
RSL = 640
PINIT = 5136
NPOS = 4992


def _mm2_body(x_ref, w0_ref, w1_ref, t0_ref, t1_ref):
    xb = x_ref[...]
    t0_ref[...] = jnp.dot(xb, w0_ref[...], preferred_element_type=jnp.float32)
    t1_ref[...] = jnp.dot(xb, w1_ref[...], preferred_element_type=jnp.float32)


def _tc_mm2(x, w0, w1):
    grid = (N // BLK,)
    return pl.pallas_call(
        _mm2_body,
        grid=grid,
        in_specs=[
            pl.BlockSpec((BLK, D), lambda i: (i, 0)),
            pl.BlockSpec((D, D), lambda i: (0, 0)),
            pl.BlockSpec((D, D), lambda i: (0, 0)),
        ],
        out_specs=[
            pl.BlockSpec((BLK, D), lambda i: (i, 0)),
            pl.BlockSpec((BLK, D), lambda i: (i, 0)),
        ],
        out_shape=[
            jax.ShapeDtypeStruct((N, D), jnp.float32),
            jax.ShapeDtypeStruct((N, D), jnp.float32),
        ],
    )(x, w0, w1)


def _mid_body(p0_ref, p1_ref, b0_ref, b1_ref, w0_ref, w1_ref, t0_ref, t1_ref):
    h = jax.nn.relu(p0_ref[...] + p1_ref[...] + b0_ref[...] + b1_ref[...])
    t0_ref[...] = jnp.dot(h, w0_ref[...], preferred_element_type=jnp.float32)
    t1_ref[...] = jnp.dot(h, w1_ref[...], preferred_element_type=jnp.float32)


def _tc_mid(p0, p1, b0, b1, w0, w1):
    grid = (N // BLK,)
    return pl.pallas_call(
        _mid_body,
        grid=grid,
        in_specs=[
            pl.BlockSpec((BLK, D), lambda i: (i, 0)),
            pl.BlockSpec((BLK, D), lambda i: (i + N // BLK, 0)),
            pl.BlockSpec((1, D), lambda i: (0, 0)),
            pl.BlockSpec((1, D), lambda i: (0, 0)),
            pl.BlockSpec((D, D), lambda i: (0, 0)),
            pl.BlockSpec((D, D), lambda i: (0, 0)),
        ],
        out_specs=[
            pl.BlockSpec((BLK, D), lambda i: (i, 0)),
            pl.BlockSpec((BLK, D), lambda i: (i, 0)),
        ],
        out_shape=[
            jax.ShapeDtypeStruct((N, D), jnp.float32),
            jax.ShapeDtypeStruct((N, D), jnp.float32),
        ],
    )(p0, p1, b0, b1, w0, w1)


def _fin_body(p0_ref, p1_ref, b0_ref, b1_ref, x_ref, o_ref):
    h = jax.nn.relu(p0_ref[...] + p1_ref[...] + b0_ref[...] + b1_ref[...])
    o_ref[...] = jax.nn.relu(h + x_ref[...])


def _tc_fin(p0, p1, b0, b1, x):
    grid = (N // BLK,)
    return pl.pallas_call(
        _fin_body,
        grid=grid,
        in_specs=[
            pl.BlockSpec((BLK, D), lambda i: (i, 0)),
            pl.BlockSpec((BLK, D), lambda i: (i + N // BLK, 0)),
            pl.BlockSpec((1, D), lambda i: (0, 0)),
            pl.BlockSpec((1, D), lambda i: (0, 0)),
            pl.BlockSpec((BLK, D), lambda i: (i, 0)),
        ],
        out_specs=pl.BlockSpec((BLK, D), lambda i: (i, 0)),
        out_shape=jax.ShapeDtypeStruct((N, D), jnp.float32),
    )(p0, p1, b0, b1, x)


def _sc_prop_body(t0_hbm, t1_hbm, pk_hbm, cf_hbm, off_hbm, p_hbm,
                  acc, rows0, rows1, pk0b, pk1b, cf0b, cf1b,
                  idx0, idx1, dstl0, dstl1, offs_v, semA0, semA1, semM):
    wid = lax.axis_index("c") * 16 + lax.axis_index("s")
    pltpu.sync_copy(off_hbm, offs_v.at[pl.ds(0, 2 * (NBUCK + 1))])
    nlast = N - (NBUCK - 1) * BROWS

    @pl.loop(0, (NITEM + 31) // 32)
    def _(k):
        item = wid + 32 * k

        @pl.when(item < NITEM)
        def _():
            r = item // NBUCK
            b = item - r * NBUCK
            zero = jnp.zeros((16,), jnp.float32)

            @pl.loop(0, BROWS * D // 256)
            def _(rr):
                for j in range(16):
                    acc[pl.ds(rr * 256 + j * 16, 16)] = zero

            ovec = offs_v[pl.ds(r * (NBUCK + 1) + b, 16)]
            lo = pl.multiple_of(ovec[0], CHUNK)
            hi = ovec[1]
            nch = (hi - lo) // CHUNK
            rbase = pl.multiple_of(r * CAP + lo, 8)

            def meta_start(ci, pkb, cfb):
                pltpu.async_copy(
                    pk_hbm.at[pl.ds(rbase + ci * CHUNK, CHUNK)], pkb, semM)
                pltpu.async_copy(
                    cf_hbm.at[pl.ds(rbase + ci * CHUNK, CHUNK)],
                    cfb.at[pl.ds(0, CHUNK)], semM)

            def meta_wait(ci, pkb, cfb):
                pltpu.make_async_copy(
                    pk_hbm.at[pl.ds(rbase + ci * CHUNK, CHUNK)], pkb,
                    semM).wait()
                pltpu.make_async_copy(
                    cf_hbm.at[pl.ds(rbase + ci * CHUNK, CHUNK)],
                    cfb.at[pl.ds(0, CHUNK)], semM).wait()

            def build(pkb, idxb, dstlb):
                for j in range(CHUNK // 16):
                    v = pkb[pl.ds(j * 16, 16)]
                    idxb[pl.ds(j * 16, 16)] = lax.shift_right_logical(v, 7)
                    dstlb[pl.ds(j * 16, 16)] = lax.bitwise_and(v, BROWS - 1)

            def gather_start(idxb, rowsb, semX):
                @pl.when(r == 0)
                def _():
                    pltpu.async_copy(t0_hbm.at[idxb], rowsb, semX)

                @pl.when(r == 1)
                def _():
                    pltpu.async_copy(t1_hbm.at[idxb], rowsb, semX)

            def gather_wait(idxb, rowsb, semX):
                @pl.when(r == 0)
                def _():
                    pltpu.make_async_copy(t0_hbm.at[idxb], rowsb, semX).wait()

                @pl.when(r == 1)
                def _():
                    pltpu.make_async_copy(t1_hbm.at[idxb], rowsb, semX).wait()

            def compute(rowsb, cfb, dstlb):
                @pl.loop(0, CHUNK // 4)
                def _(q):
                    i = q * 4
                    cs = [cfb[pl.ds(i + e, 16)][0] for e in range(4)]
                    db = [dstlb[pl.ds(i + e, 16)][0] * D for e in range(4)]
                    for j in range(D // 16):
                        for e in range(4):
                            v = rowsb[pl.ds(i + e, 1), pl.ds(j * 16, 16)]
                            plsc.addupdate(
                                acc.at[pl.ds(db[e] + j * 16, 16)],
                                (cs[e] * v).reshape(16))

            @pl.when(nch > 0)
            def _():
                meta_start(0, pk0b, cf0b)
                meta_wait(0, pk0b, cf0b)
                build(pk0b, idx0, dstl0)
                gather_start(idx0, rows0, semA0)

                @pl.when(nch > 1)
                def _():
                    meta_start(1, pk1b, cf1b)

            def stage_body(ci, pkb_n, cfb_n, idxb_n, dstlb_n, rowsb_n, semN,
                           pkb_c, cfb_c, idxb_c, dstlb_c, rowsb_c, semC):
                @pl.when(ci + 1 < nch)
                def _():
                    meta_wait(ci + 1, pkb_n, cfb_n)
                    build(pkb_n, idxb_n, dstlb_n)
                    gather_start(idxb_n, rowsb_n, semN)

                gather_wait(idxb_c, rowsb_c, semC)
                compute(rowsb_c, cfb_c, dstlb_c)

                @pl.when(ci + 2 < nch)
                def _():
                    meta_start(ci + 2, pkb_c, cfb_c)

            @pl.loop(0, nch)
            def _(ci):
                @pl.when(lax.rem(ci, 2) == 0)
                def _():
                    stage_body(ci, pk1b, cf1b, idx1, dstl1, rows1, semA1,
                               pk0b, cf0b, idx0, dstl0, rows0, semA0)

                @pl.when(lax.rem(ci, 2) == 1)
                def _():
                    stage_body(ci, pk0b, cf0b, idx0, dstl0, rows0, semA0,
                               pk1b, cf1b, idx1, dstl1, rows1, semA1)

            row0 = (r * N + b * BROWS) * D

            @pl.when(b < NBUCK - 1)
            def _():
                pltpu.sync_copy(acc, p_hbm.at[pl.ds(row0, BROWS * D)])

            @pl.when(b == NBUCK - 1)
            def _():
                pltpu.sync_copy(acc.at[pl.ds(0, nlast * D)],
                                p_hbm.at[pl.ds(row0, nlast * D)])


_sc_prop = pl.kernel(
    _sc_prop_body,
    mesh=plsc.VectorSubcoreMesh(core_axis_name="c", subcore_axis_name="s"),
    out_type=jax.ShapeDtypeStruct((2 * N * D,), jnp.float32),
    scratch_types=[
        pltpu.VMEM((BROWS * D,), jnp.float32),
        pltpu.VMEM((CHUNK, D), jnp.float32),
        pltpu.VMEM((CHUNK, D), jnp.float32),
        pltpu.VMEM((CHUNK,), jnp.int32),
        pltpu.VMEM((CHUNK,), jnp.int32),
        pltpu.VMEM((CHUNK + 16,), jnp.float32),
        pltpu.VMEM((CHUNK + 16,), jnp.float32),
        pltpu.VMEM((CHUNK,), jnp.int32),
        pltpu.VMEM((CHUNK,), jnp.int32),
        pltpu.VMEM((CHUNK + 16,), jnp.int32),
        pltpu.VMEM((CHUNK + 16,), jnp.int32),
        pltpu.VMEM((2 * (NBUCK + 1) + 16,), jnp.int32),
        pltpu.SemaphoreType.DMA,
        pltpu.SemaphoreType.DMA,
        pltpu.SemaphoreType.DMA,
    ],
)


def _quake_rsqrt(v):
    i = plsc.bitcast(v, jnp.int32)
    i = 0x5F3759DF - lax.shift_right_logical(i, 1)
    y = plsc.bitcast(i, jnp.float32)
    for _ in range(3):
        y = y * (1.5 - 0.5 * v * y * y)
    return y


ELAST = E - 15 * PER
ILAST = CAP - 15 * PINIT
RLAST = NHIST - 15 * RSL


def _sc_prep_body(s_hbm, d_hbm, pk_hbm, cf_hbm, off_hbm,
                  sv, dv, hs, hd, rbuf, cbuf, gv, lc, offv, msv,
                  cev, pkv, posv, pb0, pb1, ibuf, zbuf,
                  stage_s, stage_d, comb_s, comb_d, grid_spm,
                  cnt_s, sem, semp):
    c = lax.axis_index("c")
    t = lax.axis_index("s")
    ones16 = jnp.full((16,), 1.0, jnp.float32)
    z16f = jnp.zeros((16,), jnp.float32)
    iota16 = lax.iota(jnp.int32, 16)

    pltpu.sync_copy(s_hbm.at[pl.ds(c * E + t * PER, PER)], sv)
    pltpu.sync_copy(d_hbm.at[pl.ds(c * E + t * PER, PER)], dv)

    @pl.when(t == 15)
    def _():
        for j in range((PER - ELAST) // 16):
            sv[pl.ds(ELAST + 16 * j, 16)] = jnp.full((16,), SENT_NODE, jnp.int32)
            dv[pl.ds(ELAST + 16 * j, 16)] = jnp.full((16,), SENT_DST, jnp.int32)

    @pl.loop(0, NHIST // 16)
    def _(k):
        hs[pl.ds(k * 16, 16)] = z16f
        hd[pl.ds(k * 16, 16)] = z16f

    for k in range(6):
        lc[pl.ds(k * 16, 16)] = z16f

    @pl.loop(0, NGRP)
    def _(g):
        svv = sv[pl.ds(g * 16, 16)]
        dvv = dv[pl.ds(g * 16, 16)]
        plsc.addupdate_scatter(hs, [svv], ones16)
        plsc.addupdate_scatter(hd, [dvv], ones16)
        plsc.addupdate_scatter(lc, [lax.shift_right_logical(dvv, 7)], ones16)

    pltpu.sync_copy(hs, stage_s.at[pl.ds(t * NHIST, NHIST)])
    pltpu.sync_copy(hd, stage_d.at[pl.ds(t * NHIST, NHIST)])
    pltpu.sync_copy(lc.at[pl.ds(0, 80)], grid_spm.at[pl.ds(t * 80, 80)])
    plsc.subcore_barrier()

    def _reduce(stage, comb):
        for tt in range(16):
            pltpu.async_copy(stage.at[pl.ds(tt * NHIST + t * RSL, RSL)],
                             rbuf.at[pl.ds(tt * RSL, RSL)], sem)
        for tt in range(16):
            pltpu.make_async_copy(stage.at[pl.ds(tt * NHIST + t * RSL, RSL)],
                                  rbuf.at[pl.ds(tt * RSL, RSL)], sem).wait()

        @pl.loop(0, RSL // 16)
        def _(k):
            s = rbuf[pl.ds(k * 16, 16)]
            for tt in range(1, 16):
                s = s + rbuf[pl.ds(tt * RSL + k * 16, 16)]
            cbuf[pl.ds(k * 16, 16)] = s

        @pl.when(t < 15)
        def _():
            pltpu.sync_copy(cbuf, comb.at[pl.ds(t * RSL, RSL)])

        @pl.when(t == 15)
        def _():
            pltpu.sync_copy(cbuf.at[pl.ds(0, RLAST)],
                            comb.at[pl.ds(t * RSL, RLAST)])

    _reduce(stage_s, comb_s)
    _reduce(stage_d, comb_d)
    plsc.subcore_barrier()

    pltpu.sync_copy(comb_s, hs)
    pltpu.sync_copy(comb_d, hd)

    @pl.loop(0, NHIST // 16)
    def _(k):
        hs[pl.ds(k * 16, 16)] = _quake_rsqrt(
            jnp.maximum(hs[pl.ds(k * 16, 16)], 1.0))
        hd[pl.ds(k * 16, 16)] = _quake_rsqrt(
            jnp.maximum(hd[pl.ds(k * 16, 16)], 1.0))

    @pl.loop(0, NGRP)
    def _(g):
        svv = sv[pl.ds(g * 16, 16)]
        dvv = dv[pl.ds(g * 16, 16)]
        cev[pl.ds(g * 16, 16)] = (plsc.load_gather(hs, [svv]) *
                                  plsc.load_gather(hd, [dvv]))

    pltpu.sync_copy(grid_spm, gv.at[pl.ds(0, 1280)])
    carry = jnp.int32(0)
    for k in range(5):
        s = gv[pl.ds(k * 16, 16)]
        for tt in range(1, 16):
            s = s + gv[pl.ds(tt * 80 + k * 16, 16)]
        cnt_i = s.astype(jnp.int32)
        pad = ((cnt_i + (CHUNK - 1)) // CHUNK) * CHUNK
        if k == 4:
            pad = jnp.where(iota16 == 15, 0, pad)
        incl = plsc.cumsum(pad) + carry
        offv[pl.ds(k * 16, 16)] = incl - pad
        carry = incl[15]

    @pl.when(t == 0)
    def _():
        pltpu.sync_copy(offv.at[pl.ds(0, 80)], off_hbm.at[pl.ds(c * 80, 80)])

    for k in range(5):
        msv[pl.ds(k * 16, 16)] = offv[pl.ds(k * 16, 16)] + c * CAP

    @pl.loop(0, t)
    def _(tt):
        for k in range(5):
            msv[pl.ds(k * 16, 16)] = (msv[pl.ds(k * 16, 16)] +
                                      gv[pl.ds(tt * 80 + k * 16, 16)].astype(jnp.int32))

    for k in range(5):
        v = msv[pl.ds(k * 16, 16)]
        for lane in range(16):
            cnt_s[k * 16 + lane] = v[lane]

    @pl.loop(0, PINIT // 16)
    def _(j):
        g = t * PINIT + j * 16 + iota16
        ibuf[pl.ds(j * 16, 16)] = lax.rem(g, N) * BROWS
        zbuf[pl.ds(j * 16, 16)] = z16f

    @pl.when(t < 15)
    def _():
        pltpu.sync_copy(ibuf, pk_hbm.at[pl.ds(c * CAP + t * PINIT, PINIT)])
        pltpu.sync_copy(zbuf, cf_hbm.at[pl.ds(c * CAP + t * PINIT, PINIT)])

    @pl.when(t == 15)
    def _():
        pltpu.sync_copy(ibuf.at[pl.ds(0, ILAST)],
                        pk_hbm.at[pl.ds(c * CAP + 15 * PINIT, ILAST)])
        pltpu.sync_copy(zbuf.at[pl.ds(0, ILAST)],
                        cf_hbm.at[pl.ds(c * CAP + 15 * PINIT, ILAST)])

    plsc.subcore_barrier()

    @pl.loop(0, NPOS // 16)
    def _(j):
        posv[pl.ds(j * 16, 16)] = (c * CAP + DATA_CAP +
                                   lax.rem(j * 16 + iota16, 256))

    @pl.loop(0, NGRP)
    def _(g):
        dvv = dv[pl.ds(g * 16, 16)]
        bv = lax.shift_right_logical(dvv, 7)
        pv = jnp.zeros((16,), jnp.int32)
        for lane in range(16):
            b = bv[lane]
            p = cnt_s[b]
            cnt_s[b] = p + 1
            pv = jnp.where(iota16 == lane, p, pv)
        posv[pl.ds(g * 16, 16)] = pv
        pkv[pl.ds(g * 16, 16)] = (sv[pl.ds(g * 16, 16)] * BROWS +
                                  lax.bitwise_and(dvv, BROWS - 1))

    nsc = NPOS // 128
    for j in range(nsc):
        pb = pb0 if j % 2 == 0 else pb1
        if j >= 2:
            pltpu.make_async_copy(pkv.at[pl.ds((j - 2) * 128, 128)],
                                  pk_hbm.at[pb], semp).wait()
            pltpu.make_async_copy(cev.at[pl.ds((j - 2) * 128, 128)],
                                  cf_hbm.at[pb], semp).wait()
        for k in range(8):
            pb[pl.ds(k * 16, 16)] = posv[pl.ds(j * 128 + k * 16, 16)]
        pltpu.async_copy(pkv.at[pl.ds(j * 128, 128)], pk_hbm.at[pb], semp)
        pltpu.async_copy(cev.at[pl.ds(j * 128, 128)], cf_hbm.at[pb], semp)
    for j in range(nsc - 2, nsc):
        pb = pb0 if j % 2 == 0 else pb1
        pltpu.make_async_copy(pkv.at[pl.ds(j * 128, 128)],
                              pk_hbm.at[pb], semp).wait()
        pltpu.make_async_copy(cev.at[pl.ds(j * 128, 128)],
                              cf_hbm.at[pb], semp).wait()


_sc_cp = pltpu.CompilerParams()
if "needs_layout_passes" in pltpu.CompilerParams.__dataclass_fields__:
    _sc_cp = dataclasses.replace(_sc_cp, needs_layout_passes=False)

_sc_prep = pl.kernel(
    _sc_prep_body,
    mesh=plsc.VectorSubcoreMesh(core_axis_name="c", subcore_axis_name="s"),
    compiler_params=_sc_cp,
    out_type=[
        jax.ShapeDtypeStruct((2 * CAP,), jnp.int32),
        jax.ShapeDtypeStruct((2 * CAP,), jnp.float32),
        jax.ShapeDtypeStruct((2 * (NBUCK + 1),), jnp.int32),
    ],
    scratch_types=[
        pltpu.VMEM((PER,), jnp.int32),
        pltpu.VMEM((PER,), jnp.int32),
        pltpu.VMEM((NHIST,), jnp.float32),
        pltpu.VMEM((NHIST,), jnp.float32),
        pltpu.VMEM((16 * RSL,), jnp.float32),
        pltpu.VMEM((RSL,), jnp.float32),
        pltpu.VMEM((1280 + 16,), jnp.float32),
        pltpu.VMEM((96,), jnp.float32),
        pltpu.VMEM((96,), jnp.int32),
        pltpu.VMEM((96,), jnp.int32),
        pltpu.VMEM((NPOS,), jnp.float32),
        pltpu.VMEM((NPOS,), jnp.int32),
        pltpu.VMEM((NPOS,), jnp.int32),
        pltpu.VMEM((128,), jnp.int32),
        pltpu.VMEM((128,), jnp.int32),
        pltpu.VMEM((PINIT,), jnp.int32),
        pltpu.VMEM((PINIT,), jnp.float32),
        pltpu.VMEM_SHARED((16 * NHIST + 112,), jnp.float32),
        pltpu.VMEM_SHARED((16 * NHIST + 112,), jnp.float32),
        pltpu.VMEM_SHARED((NHIST,), jnp.float32),
        pltpu.VMEM_SHARED((NHIST,), jnp.float32),
        pltpu.VMEM_SHARED((1280,), jnp.float32),
        pltpu.SMEM((96,), jnp.int32),
        pltpu.SemaphoreType.DMA,
        pltpu.SemaphoreType.DMA,
    ],
)


def _prep_jax(ei):
    src, dst = ei[0], ei[1]
    E = src.shape[0]
    ones = jnp.ones((E,), jnp.float32)
    deg_out = jax.ops.segment_sum(ones, src, num_segments=N)
    deg_in = jax.ops.segment_sum(ones, dst, num_segments=N)
    ns = jax.lax.rsqrt(jnp.maximum(deg_out, 1.0))
    nd = jax.lax.rsqrt(jnp.maximum(deg_in, 1.0))
    coef = ns[src] * nd[dst]
    bucket = dst // BROWS
    order = jnp.argsort(bucket, stable=True)
    srcs = src[order]
    dsts = dst[order]
    coefs = coef[order]
    cnt = jax.ops.segment_sum(jnp.ones((E,), jnp.int32), bucket,
                              num_segments=NBUCK)
    padded = ((cnt + CHUNK - 1) // CHUNK) * CHUNK
    off = jnp.concatenate(
        [jnp.zeros((1,), jnp.int32), jnp.cumsum(padded, dtype=jnp.int32)])
    start = jnp.cumsum(cnt) - cnt
    bsort = bucket[order]
    rank = jnp.arange(E, dtype=jnp.int32) - start[bsort]
    pos = off[bsort] + rank
    packed = (jnp.arange(CAP, dtype=jnp.int32) % N) * BROWS
    packed = packed.at[pos].set(srcs * BROWS + (dsts % BROWS))
    coefa = jnp.zeros((CAP,), jnp.float32).at[pos].set(coefs)
    return packed, coefa, off


def kernel(x, edge_index_rel0, edge_index_rel1, W00, b00, W01, b01, W10, b10, W11, b11):
    b00 = b00.reshape(1, D)
    b01 = b01.reshape(1, D)
    b10 = b10.reshape(1, D)
    b11 = b11.reshape(1, D)
    zpad = jnp.zeros((80,), jnp.int32)
    s_all = jnp.concatenate([edge_index_rel0[0], edge_index_rel1[0], zpad])
    d_all = jnp.concatenate([edge_index_rel0[1], edge_index_rel1[1], zpad])
    packed, coef, off = _sc_prep(s_all, d_all)
    t0, t1 = _tc_mm2(x, W00, W01)
    p_a = _sc_prop(t0, t1, packed, coef, off).reshape(2 * N, D)
    t2, t3 = _tc_mid(p_a, p_a, b00, b01, W10, W11)
    p_b = _sc_prop(t2, t3, packed, coef, off).reshape(2 * N, D)
    return _tc_fin(p_b, p_b, b10, b11, x)


# speedup vs baseline: 1.6359x; 1.6359x over previous
import dataclasses
import functools

import jax
import jax.numpy as jnp
from jax import lax
from jax.experimental import pallas as pl
from jax.experimental.pallas import tpu as pltpu
from jax.experimental.pallas import tpu_sc as plsc

N = 10000
D = 512
BLK = 400

BROWS = 128
NBUCK = (N + BROWS - 1) // BROWS
NITEM = 2 * NBUCK
CHUNK = 48
E = 78000
DATA_CAP = 81696
CAP = DATA_CAP + 256
PER = 4880
NGRP = PER // 16
NHIST = 10128
SENT_NODE = 10008
SENT_DST = 10112
RSL = 640
PINIT = 5136
NPOS = 4992


def _mm2_body(x_ref, w0_ref, w1_ref, t0_ref, t1_ref):
    xb = x_ref[...]
    t0_ref[...] = jnp.dot(xb, w0_ref[...], preferred_element_type=jnp.float32)
    t1_ref[...] = jnp.dot(xb, w1_ref[...], preferred_element_type=jnp.float32)


def _tc_mm2(x, w0, w1):
    grid = (N // BLK,)
    return pl.pallas_call(
        _mm2_body,
        grid=grid,
        in_specs=[
            pl.BlockSpec((BLK, D), lambda i: (i, 0)),
            pl.BlockSpec((D, D), lambda i: (0, 0)),
            pl.BlockSpec((D, D), lambda i: (0, 0)),
        ],
        out_specs=[
            pl.BlockSpec((BLK, D), lambda i: (i, 0)),
            pl.BlockSpec((BLK, D), lambda i: (i, 0)),
        ],
        out_shape=[
            jax.ShapeDtypeStruct((N, D), jnp.float32),
            jax.ShapeDtypeStruct((N, D), jnp.float32),
        ],
    )(x, w0, w1)


def _mid_body(p0_ref, p1_ref, b0_ref, b1_ref, w0_ref, w1_ref, t0_ref, t1_ref):
    h = jax.nn.relu(p0_ref[...] + p1_ref[...] + b0_ref[...] + b1_ref[...])
    t0_ref[...] = jnp.dot(h, w0_ref[...], preferred_element_type=jnp.float32)
    t1_ref[...] = jnp.dot(h, w1_ref[...], preferred_element_type=jnp.float32)


def _tc_mid(p0, p1, b0, b1, w0, w1):
    grid = (N // BLK,)
    return pl.pallas_call(
        _mid_body,
        grid=grid,
        in_specs=[
            pl.BlockSpec((BLK, D), lambda i: (i, 0)),
            pl.BlockSpec((BLK, D), lambda i: (i + N // BLK, 0)),
            pl.BlockSpec((1, D), lambda i: (0, 0)),
            pl.BlockSpec((1, D), lambda i: (0, 0)),
            pl.BlockSpec((D, D), lambda i: (0, 0)),
            pl.BlockSpec((D, D), lambda i: (0, 0)),
        ],
        out_specs=[
            pl.BlockSpec((BLK, D), lambda i: (i, 0)),
            pl.BlockSpec((BLK, D), lambda i: (i, 0)),
        ],
        out_shape=[
            jax.ShapeDtypeStruct((N, D), jnp.float32),
            jax.ShapeDtypeStruct((N, D), jnp.float32),
        ],
    )(p0, p1, b0, b1, w0, w1)


def _fin_body(p0_ref, p1_ref, b0_ref, b1_ref, x_ref, o_ref):
    h = jax.nn.relu(p0_ref[...] + p1_ref[...] + b0_ref[...] + b1_ref[...])
    o_ref[...] = jax.nn.relu(h + x_ref[...])


def _tc_fin(p0, p1, b0, b1, x):
    grid = (N // BLK,)
    return pl.pallas_call(
        _fin_body,
        grid=grid,
        in_specs=[
            pl.BlockSpec((BLK, D), lambda i: (i, 0)),
            pl.BlockSpec((BLK, D), lambda i: (i + N // BLK, 0)),
            pl.BlockSpec((1, D), lambda i: (0, 0)),
            pl.BlockSpec((1, D), lambda i: (0, 0)),
            pl.BlockSpec((BLK, D), lambda i: (i, 0)),
        ],
        out_specs=pl.BlockSpec((BLK, D), lambda i: (i, 0)),
        out_shape=jax.ShapeDtypeStruct((N, D), jnp.float32),
    )(p0, p1, b0, b1, x)


def _sc_prop_body(t0_hbm, t1_hbm, pk_hbm, cf_hbm, off_hbm, p_hbm,
                  acc, rows0, rows1, pk0b, pk1b, cf0b, cf1b,
                  idx0, idx1, dstl0, dstl1, offs_v, semA0, semA1, semM):
    wid = lax.axis_index("c") * 16 + lax.axis_index("s")
    pltpu.sync_copy(off_hbm, offs_v.at[pl.ds(0, 2 * (NBUCK + 1))])
    nlast = N - (NBUCK - 1) * BROWS

    @pl.loop(0, (NITEM + 31) // 32)
    def _(k):
        item = wid + 32 * k

        @pl.when(item < NITEM)
        def _():
            r = item // NBUCK
            b = item - r * NBUCK
            zero = jnp.zeros((16,), jnp.float32)

            @plsc.parallel_loop(0, BROWS * D // 256, 1, unroll=2)
            def _(rr):
                for j in range(16):
                    acc[pl.ds(rr * 256 + j * 16, 16)] = zero

            ovec = offs_v[pl.ds(r * (NBUCK + 1) + b, 16)]
            lo = pl.multiple_of(ovec[0], CHUNK)
            hi = ovec[1]
            nch = (hi - lo) // CHUNK
            rbase = pl.multiple_of(r * CAP + lo, 8)

            def meta_start(ci, pkb, cfb):
                pltpu.async_copy(
                    pk_hbm.at[pl.ds(rbase + ci * CHUNK, CHUNK)], pkb, semM)
                pltpu.async_copy(
                    cf_hbm.at[pl.ds(rbase + ci * CHUNK, CHUNK)],
                    cfb.at[pl.ds(0, CHUNK)], semM)

            def meta_wait(ci, pkb, cfb):
                pltpu.make_async_copy(
                    pk_hbm.at[pl.ds(rbase + ci * CHUNK, CHUNK)], pkb,
                    semM).wait()
                pltpu.make_async_copy(
                    cf_hbm.at[pl.ds(rbase + ci * CHUNK, CHUNK)],
                    cfb.at[pl.ds(0, CHUNK)], semM).wait()

            def build(pkb, idxb, dstlb):
                for j in range(CHUNK // 16):
                    v = pkb[pl.ds(j * 16, 16)]
                    idxb[pl.ds(j * 16, 16)] = lax.shift_right_logical(v, 7)
                    dstlb[pl.ds(j * 16, 16)] = lax.bitwise_and(v, BROWS - 1)

            def gather_start(idxb, rowsb, semX):
                @pl.when(r == 0)
                def _():
                    pltpu.async_copy(t0_hbm.at[idxb], rowsb, semX)

                @pl.when(r == 1)
                def _():
                    pltpu.async_copy(t1_hbm.at[idxb], rowsb, semX)

            def gather_wait(idxb, rowsb, semX):
                @pl.when(r == 0)
                def _():
                    pltpu.make_async_copy(t0_hbm.at[idxb], rowsb, semX).wait()

                @pl.when(r == 1)
                def _():
                    pltpu.make_async_copy(t1_hbm.at[idxb], rowsb, semX).wait()

            def compute(rowsb, cfb, dstlb):
                @plsc.parallel_loop(0, CHUNK, 1, unroll=4)
                def _(i):
                    cs = cfb[pl.ds(i, 16)][0]
                    dbase = dstlb[pl.ds(i, 16)][0] * D
                    for j in range(D // 16):
                        v = rowsb[pl.ds(i, 1), pl.ds(j * 16, 16)]
                        plsc.addupdate(
                            acc.at[pl.ds(dbase + j * 16, 16)],
                            (cs * v).reshape(16))

            @pl.when(nch > 0)
            def _():
                meta_start(0, pk0b, cf0b)
                meta_wait(0, pk0b, cf0b)
                build(pk0b, idx0, dstl0)
                gather_start(idx0, rows0, semA0)

                @pl.when(nch > 1)
                def _():
                    meta_start(1, pk1b, cf1b)

            def stage_body(ci, pkb_n, cfb_n, idxb_n, dstlb_n, rowsb_n, semN,
                           pkb_c, cfb_c, idxb_c, dstlb_c, rowsb_c, semC):
                @pl.when(ci + 1 < nch)
                def _():
                    meta_wait(ci + 1, pkb_n, cfb_n)
                    build(pkb_n, idxb_n, dstlb_n)
                    gather_start(idxb_n, rowsb_n, semN)

                gather_wait(idxb_c, rowsb_c, semC)
                compute(rowsb_c, cfb_c, dstlb_c)

                @pl.when(ci + 2 < nch)
                def _():
                    meta_start(ci + 2, pkb_c, cfb_c)

            @pl.loop(0, nch)
            def _(ci):
                @pl.when(lax.rem(ci, 2) == 0)
                def _():
                    stage_body(ci, pk1b, cf1b, idx1, dstl1, rows1, semA1,
                               pk0b, cf0b, idx0, dstl0, rows0, semA0)

                @pl.when(lax.rem(ci, 2) == 1)
                def _():
                    stage_body(ci, pk0b, cf0b, idx0, dstl0, rows0, semA0,
                               pk1b, cf1b, idx1, dstl1, rows1, semA1)

            row0 = (r * N + b * BROWS) * D

            @pl.when(b < NBUCK - 1)
            def _():
                pltpu.sync_copy(acc, p_hbm.at[pl.ds(row0, BROWS * D)])

            @pl.when(b == NBUCK - 1)
            def _():
                pltpu.sync_copy(acc.at[pl.ds(0, nlast * D)],
                                p_hbm.at[pl.ds(row0, nlast * D)])


_sc_prop = pl.kernel(
    _sc_prop_body,
    mesh=plsc.VectorSubcoreMesh(core_axis_name="c", subcore_axis_name="s"),
    out_type=jax.ShapeDtypeStruct((2 * N * D,), jnp.float32),
    scratch_types=[
        pltpu.VMEM((BROWS * D,), jnp.float32),
        pltpu.VMEM((CHUNK, D), jnp.float32),
        pltpu.VMEM((CHUNK, D), jnp.float32),
        pltpu.VMEM((CHUNK,), jnp.int32),
        pltpu.VMEM((CHUNK,), jnp.int32),
        pltpu.VMEM((CHUNK + 16,), jnp.float32),
        pltpu.VMEM((CHUNK + 16,), jnp.float32),
        pltpu.VMEM((CHUNK,), jnp.int32),
        pltpu.VMEM((CHUNK,), jnp.int32),
        pltpu.VMEM((CHUNK + 16,), jnp.int32),
        pltpu.VMEM((CHUNK + 16,), jnp.int32),
        pltpu.VMEM((2 * (NBUCK + 1) + 16,), jnp.int32),
        pltpu.SemaphoreType.DMA,
        pltpu.SemaphoreType.DMA,
        pltpu.SemaphoreType.DMA,
    ],
)


def _quake_rsqrt(v):
    i = plsc.bitcast(v, jnp.int32)
    i = 0x5F3759DF - lax.shift_right_logical(i, 1)
    y = plsc.bitcast(i, jnp.float32)
    for _ in range(3):
        y = y * (1.5 - 0.5 * v * y * y)
    return y


ELAST = E - 15 * PER
ILAST = CAP - 15 * PINIT
RLAST = NHIST - 15 * RSL


def _sc_prep_body(s_hbm, d_hbm, pk_hbm, cf_hbm, off_hbm,
                  sv, dv, hs, hd, rbuf, cbuf, gv, lc, offv, msv,
                  cev, pkv, posv, pb0, pb1, ibuf, zbuf,
                  stage_s, stage_d, comb_s, comb_d, grid_spm,
                  cnt_s, sem, semp):
    c = lax.axis_index("c")
    t = lax.axis_index("s")
    ones16 = jnp.full((16,), 1.0, jnp.float32)
    z16f = jnp.zeros((16,), jnp.float32)
    iota16 = lax.iota(jnp.int32, 16)

    pltpu.sync_copy(s_hbm.at[pl.ds(c * E + t * PER, PER)], sv)
    pltpu.sync_copy(d_hbm.at[pl.ds(c * E + t * PER, PER)], dv)

    @pl.when(t == 15)
    def _():
        for j in range((PER - ELAST) // 16):
            sv[pl.ds(ELAST + 16 * j, 16)] = jnp.full((16,), SENT_NODE, jnp.int32)
            dv[pl.ds(ELAST + 16 * j, 16)] = jnp.full((16,), SENT_DST, jnp.int32)

    @pl.loop(0, NHIST // 16)
    def _(k):
        hs[pl.ds(k * 16, 16)] = z16f
        hd[pl.ds(k * 16, 16)] = z16f

    for k in range(6):
        lc[pl.ds(k * 16, 16)] = z16f

    @pl.loop(0, NGRP)
    def _(g):
        svv = sv[pl.ds(g * 16, 16)]
        dvv = dv[pl.ds(g * 16, 16)]
        plsc.addupdate_scatter(hs, [svv], ones16)
        plsc.addupdate_scatter(hd, [dvv], ones16)
        plsc.addupdate_scatter(lc, [lax.shift_right_logical(dvv, 7)], ones16)

    pltpu.sync_copy(hs, stage_s.at[pl.ds(t * NHIST, NHIST)])
    pltpu.sync_copy(hd, stage_d.at[pl.ds(t * NHIST, NHIST)])
    pltpu.sync_copy(lc.at[pl.ds(0, 80)], grid_spm.at[pl.ds(t * 80, 80)])
    plsc.subcore_barrier()

    def _reduce(stage, comb):
        for tt in range(16):
            pltpu.async_copy(stage.at[pl.ds(tt * NHIST + t * RSL, RSL)],
                             rbuf.at[pl.ds(tt * RSL, RSL)], sem)
        for tt in range(16):
            pltpu.make_async_copy(stage.at[pl.ds(tt * NHIST + t * RSL, RSL)],
                                  rbuf.at[pl.ds(tt * RSL, RSL)], sem).wait()

        @pl.loop(0, RSL // 16)
        def _(k):
            s = rbuf[pl.ds(k * 16, 16)]
            for tt in range(1, 16):
                s = s + rbuf[pl.ds(tt * RSL + k * 16, 16)]
            cbuf[pl.ds(k * 16, 16)] = s

        @pl.when(t < 15)
        def _():
            pltpu.sync_copy(cbuf, comb.at[pl.ds(t * RSL, RSL)])

        @pl.when(t == 15)
        def _():
            pltpu.sync_copy(cbuf.at[pl.ds(0, RLAST)],
                            comb.at[pl.ds(t * RSL, RLAST)])

    _reduce(stage_s, comb_s)
    _reduce(stage_d, comb_d)
    plsc.subcore_barrier()

    pltpu.sync_copy(comb_s, hs)
    pltpu.sync_copy(comb_d, hd)

    @pl.loop(0, NHIST // 16)
    def _(k):
        hs[pl.ds(k * 16, 16)] = _quake_rsqrt(
            jnp.maximum(hs[pl.ds(k * 16, 16)], 1.0))
        hd[pl.ds(k * 16, 16)] = _quake_rsqrt(
            jnp.maximum(hd[pl.ds(k * 16, 16)], 1.0))

    @pl.loop(0, NGRP)
    def _(g):
        svv = sv[pl.ds(g * 16, 16)]
        dvv = dv[pl.ds(g * 16, 16)]
        cev[pl.ds(g * 16, 16)] = (plsc.load_gather(hs, [svv]) *
                                  plsc.load_gather(hd, [dvv]))

    pltpu.sync_copy(grid_spm, gv.at[pl.ds(0, 1280)])
    carry = jnp.int32(0)
    for k in range(5):
        s = gv[pl.ds(k * 16, 16)]
        for tt in range(1, 16):
            s = s + gv[pl.ds(tt * 80 + k * 16, 16)]
        cnt_i = s.astype(jnp.int32)
        pad = ((cnt_i + (CHUNK - 1)) // CHUNK) * CHUNK
        if k == 4:
            pad = jnp.where(iota16 == 15, 0, pad)
        incl = plsc.cumsum(pad) + carry
        offv[pl.ds(k * 16, 16)] = incl - pad
        carry = incl[15]

    @pl.when(t == 0)
    def _():
        pltpu.sync_copy(offv.at[pl.ds(0, 80)], off_hbm.at[pl.ds(c * 80, 80)])

    for k in range(5):
        msv[pl.ds(k * 16, 16)] = offv[pl.ds(k * 16, 16)] + c * CAP

    @pl.loop(0, t)
    def _(tt):
        for k in range(5):
            msv[pl.ds(k * 16, 16)] = (msv[pl.ds(k * 16, 16)] +
                                      gv[pl.ds(tt * 80 + k * 16, 16)].astype(jnp.int32))

    for k in range(5):
        v = msv[pl.ds(k * 16, 16)]
        for lane in range(16):
            cnt_s[k * 16 + lane] = v[lane]

    @pl.loop(0, PINIT // 16)
    def _(j):
        g = t * PINIT + j * 16 + iota16
        ibuf[pl.ds(j * 16, 16)] = lax.rem(g, N) * BROWS
        zbuf[pl.ds(j * 16, 16)] = z16f

    @pl.when(t < 15)
    def _():
        pltpu.sync_copy(ibuf, pk_hbm.at[pl.ds(c * CAP + t * PINIT, PINIT)])
        pltpu.sync_copy(zbuf, cf_hbm.at[pl.ds(c * CAP + t * PINIT, PINIT)])

    @pl.when(t == 15)
    def _():
        pltpu.sync_copy(ibuf.at[pl.ds(0, ILAST)],
                        pk_hbm.at[pl.ds(c * CAP + 15 * PINIT, ILAST)])
        pltpu.sync_copy(zbuf.at[pl.ds(0, ILAST)],
                        cf_hbm.at[pl.ds(c * CAP + 15 * PINIT, ILAST)])

    plsc.subcore_barrier()

    @pl.loop(0, NPOS // 16)
    def _(j):
        posv[pl.ds(j * 16, 16)] = (c * CAP + DATA_CAP +
                                   lax.rem(j * 16 + iota16, 256))

    @pl.loop(0, NGRP)
    def _(g):
        dvv = dv[pl.ds(g * 16, 16)]
        bv = lax.shift_right_logical(dvv, 7)
        pv = jnp.zeros((16,), jnp.int32)
        for lane in range(16):
            b = bv[lane]
            p = cnt_s[b]
            cnt_s[b] = p + 1
            pv = jnp.where(iota16 == lane, p, pv)
        posv[pl.ds(g * 16, 16)] = pv
        pkv[pl.ds(g * 16, 16)] = (sv[pl.ds(g * 16, 16)] * BROWS +
                                  lax.bitwise_and(dvv, BROWS - 1))

    nsc = NPOS // 128
    for j in range(nsc):
        pb = pb0 if j % 2 == 0 else pb1
        if j >= 2:
            pltpu.make_async_copy(pkv.at[pl.ds((j - 2) * 128, 128)],
                                  pk_hbm.at[pb], semp).wait()
            pltpu.make_async_copy(cev.at[pl.ds((j - 2) * 128, 128)],
                                  cf_hbm.at[pb], semp).wait()
        for k in range(8):
            pb[pl.ds(k * 16, 16)] = posv[pl.ds(j * 128 + k * 16, 16)]
        pltpu.async_copy(pkv.at[pl.ds(j * 128, 128)], pk_hbm.at[pb], semp)
        pltpu.async_copy(cev.at[pl.ds(j * 128, 128)], cf_hbm.at[pb], semp)
    for j in range(nsc - 2, nsc):
        pb = pb0 if j % 2 == 0 else pb1
        pltpu.make_async_copy(pkv.at[pl.ds(j * 128, 128)],
                              pk_hbm.at[pb], semp).wait()
        pltpu.make_async_copy(cev.at[pl.ds(j * 128, 128)],
                              cf_hbm.at[pb], semp).wait()


_sc_cp = pltpu.CompilerParams()
if "needs_layout_passes" in pltpu.CompilerParams.__dataclass_fields__:
    _sc_cp = dataclasses.replace(_sc_cp, needs_layout_passes=False)

_sc_prep = pl.kernel(
    _sc_prep_body,
    mesh=plsc.VectorSubcoreMesh(core_axis_name="c", subcore_axis_name="s"),
    compiler_params=_sc_cp,
    out_type=[
        jax.ShapeDtypeStruct((2 * CAP,), jnp.int32),
        jax.ShapeDtypeStruct((2 * CAP,), jnp.float32),
        jax.ShapeDtypeStruct((2 * (NBUCK + 1),), jnp.int32),
    ],
    scratch_types=[
        pltpu.VMEM((PER,), jnp.int32),
        pltpu.VMEM((PER,), jnp.int32),
        pltpu.VMEM((NHIST,), jnp.float32),
        pltpu.VMEM((NHIST,), jnp.float32),
        pltpu.VMEM((16 * RSL,), jnp.float32),
        pltpu.VMEM((RSL,), jnp.float32),
        pltpu.VMEM((1280 + 16,), jnp.float32),
        pltpu.VMEM((96,), jnp.float32),
        pltpu.VMEM((96,), jnp.int32),
        pltpu.VMEM((96,), jnp.int32),
        pltpu.VMEM((NPOS,), jnp.float32),
        pltpu.VMEM((NPOS,), jnp.int32),
        pltpu.VMEM((NPOS,), jnp.int32),
        pltpu.VMEM((128,), jnp.int32),
        pltpu.VMEM((128,), jnp.int32),
        pltpu.VMEM((PINIT,), jnp.int32),
        pltpu.VMEM((PINIT,), jnp.float32),
        pltpu.VMEM_SHARED((16 * NHIST + 112,), jnp.float32),
        pltpu.VMEM_SHARED((16 * NHIST + 112,), jnp.float32),
        pltpu.VMEM_SHARED((NHIST,), jnp.float32),
        pltpu.VMEM_SHARED((NHIST,), jnp.float32),
        pltpu.VMEM_SHARED((1280,), jnp.float32),
        pltpu.SMEM((96,), jnp.int32),
        pltpu.SemaphoreType.DMA,
        pltpu.SemaphoreType.DMA,
    ],
)


def _prep_jax(ei):
    src, dst = ei[0], ei[1]
    E = src.shape[0]
    ones = jnp.ones((E,), jnp.float32)
    deg_out = jax.ops.segment_sum(ones, src, num_segments=N)
    deg_in = jax.ops.segment_sum(ones, dst, num_segments=N)
    ns = jax.lax.rsqrt(jnp.maximum(deg_out, 1.0))
    nd = jax.lax.rsqrt(jnp.maximum(deg_in, 1.0))
    coef = ns[src] * nd[dst]
    bucket = dst // BROWS
    order = jnp.argsort(bucket, stable=True)
    srcs = src[order]
    dsts = dst[order]
    coefs = coef[order]
    cnt = jax.ops.segment_sum(jnp.ones((E,), jnp.int32), bucket,
                              num_segments=NBUCK)
    padded = ((cnt + CHUNK - 1) // CHUNK) * CHUNK
    off = jnp.concatenate(
        [jnp.zeros((1,), jnp.int32), jnp.cumsum(padded, dtype=jnp.int32)])
    start = jnp.cumsum(cnt) - cnt
    bsort = bucket[order]
    rank = jnp.arange(E, dtype=jnp.int32) - start[bsort]
    pos = off[bsort] + rank
    packed = (jnp.arange(CAP, dtype=jnp.int32) % N) * BROWS
    packed = packed.at[pos].set(srcs * BROWS + (dsts % BROWS))
    coefa = jnp.zeros((CAP,), jnp.float32).at[pos].set(coefs)
    return packed, coefa, off


def kernel(x, edge_index_rel0, edge_index_rel1, W00, b00, W01, b01, W10, b10, W11, b11):
    b00 = b00.reshape(1, D)
    b01 = b01.reshape(1, D)
    b10 = b10.reshape(1, D)
    b11 = b11.reshape(1, D)
    zpad = jnp.zeros((80,), jnp.int32)
    s_all = jnp.concatenate([edge_index_rel0[0], edge_index_rel1[0], zpad])
    d_all = jnp.concatenate([edge_index_rel0[1], edge_index_rel1[1], zpad])
    packed, coef, off = _sc_prep(s_all, d_all)
    t0, t1 = _tc_mm2(x, W00, W01)
    p_a = _sc_prop(t0, t1, packed, coef, off).reshape(2 * N, D)
    t2, t3 = _tc_mid(p_a, p_a, b00, b01, W10, W11)
    p_b = _sc_prop(t2, t3, packed, coef, off).reshape(2 * N, D)
    return _tc_fin(p_b, p_b, b10, b11, x)


# speedup vs baseline: 1.6496x; 1.0084x over previous
import dataclasses
import functools

import jax
import jax.numpy as jnp
from jax import lax
from jax.experimental import pallas as pl
from jax.experimental.pallas import tpu as pltpu
from jax.experimental.pallas import tpu_sc as plsc

N = 10000
D = 512
BLK = 400

BROWS = 128
NBUCK = (N + BROWS - 1) // BROWS
NITEM = 2 * NBUCK
CHUNK = 48
E = 78000
DATA_CAP = 81696
CAP = DATA_CAP + 256
PER = 4880
NGRP = PER // 16
NHIST = 10128
SENT_NODE = 10008
SENT_DST = 10112
RSL = 640
PINIT = 5136
NPOS = 4992


def _mm2_body(x_ref, w0_ref, w1_ref, t0_ref, t1_ref):
    xb = x_ref[...]
    t0_ref[...] = jnp.dot(xb, w0_ref[...], preferred_element_type=jnp.float32)
    t1_ref[...] = jnp.dot(xb, w1_ref[...], preferred_element_type=jnp.float32)


def _tc_mm2(x, w0, w1):
    grid = (N // BLK,)
    return pl.pallas_call(
        _mm2_body,
        grid=grid,
        in_specs=[
            pl.BlockSpec((BLK, D), lambda i: (i, 0)),
            pl.BlockSpec((D, D), lambda i: (0, 0)),
            pl.BlockSpec((D, D), lambda i: (0, 0)),
        ],
        out_specs=[
            pl.BlockSpec((BLK, D), lambda i: (i, 0)),
            pl.BlockSpec((BLK, D), lambda i: (i, 0)),
        ],
        out_shape=[
            jax.ShapeDtypeStruct((N, D), jnp.float32),
            jax.ShapeDtypeStruct((N, D), jnp.float32),
        ],
    )(x, w0, w1)


def _mid_body(p0_ref, p1_ref, b0_ref, b1_ref, w0_ref, w1_ref, t0_ref, t1_ref):
    h = jax.nn.relu(p0_ref[...] + p1_ref[...] + b0_ref[...] + b1_ref[...])
    t0_ref[...] = jnp.dot(h, w0_ref[...], preferred_element_type=jnp.float32)
    t1_ref[...] = jnp.dot(h, w1_ref[...], preferred_element_type=jnp.float32)


def _tc_mid(p0, p1, b0, b1, w0, w1):
    grid = (N // BLK,)
    return pl.pallas_call(
        _mid_body,
        grid=grid,
        in_specs=[
            pl.BlockSpec((BLK, D), lambda i: (i, 0)),
            pl.BlockSpec((BLK, D), lambda i: (i + N // BLK, 0)),
            pl.BlockSpec((1, D), lambda i: (0, 0)),
            pl.BlockSpec((1, D), lambda i: (0, 0)),
            pl.BlockSpec((D, D), lambda i: (0, 0)),
            pl.BlockSpec((D, D), lambda i: (0, 0)),
        ],
        out_specs=[
            pl.BlockSpec((BLK, D), lambda i: (i, 0)),
            pl.BlockSpec((BLK, D), lambda i: (i, 0)),
        ],
        out_shape=[
            jax.ShapeDtypeStruct((N, D), jnp.float32),
            jax.ShapeDtypeStruct((N, D), jnp.float32),
        ],
    )(p0, p1, b0, b1, w0, w1)


def _fin_body(p0_ref, p1_ref, b0_ref, b1_ref, x_ref, o_ref):
    h = jax.nn.relu(p0_ref[...] + p1_ref[...] + b0_ref[...] + b1_ref[...])
    o_ref[...] = jax.nn.relu(h + x_ref[...])


def _tc_fin(p0, p1, b0, b1, x):
    grid = (N // BLK,)
    return pl.pallas_call(
        _fin_body,
        grid=grid,
        in_specs=[
            pl.BlockSpec((BLK, D), lambda i: (i, 0)),
            pl.BlockSpec((BLK, D), lambda i: (i + N // BLK, 0)),
            pl.BlockSpec((1, D), lambda i: (0, 0)),
            pl.BlockSpec((1, D), lambda i: (0, 0)),
            pl.BlockSpec((BLK, D), lambda i: (i, 0)),
        ],
        out_specs=pl.BlockSpec((BLK, D), lambda i: (i, 0)),
        out_shape=jax.ShapeDtypeStruct((N, D), jnp.float32),
    )(p0, p1, b0, b1, x)


def _sc_prop_body(t0_hbm, t1_hbm, pk_hbm, cf_hbm, off_hbm, p_hbm,
                  acc, rows0, rows1, pk0b, pk1b, cf0b, cf1b,
                  idx0, idx1, dstl0, dstl1, offs_v, semA0, semA1, semM):
    wid = lax.axis_index("c") * 16 + lax.axis_index("s")
    pltpu.sync_copy(off_hbm, offs_v.at[pl.ds(0, 2 * (NBUCK + 1))])
    nlast = N - (NBUCK - 1) * BROWS

    @pl.loop(0, (NITEM + 31) // 32)
    def _(k):
        item = wid + 32 * k

        @pl.when(item < NITEM)
        def _():
            r = item // NBUCK
            b = item - r * NBUCK
            zero = jnp.zeros((16,), jnp.float32)

            @plsc.parallel_loop(0, BROWS * D // 256, 1, unroll=2)
            def _(rr):
                for j in range(16):
                    acc[pl.ds(rr * 256 + j * 16, 16)] = zero

            ovec = offs_v[pl.ds(r * (NBUCK + 1) + b, 16)]
            lo = pl.multiple_of(ovec[0], CHUNK)
            hi = ovec[1]
            nch = (hi - lo) // CHUNK
            rbase = pl.multiple_of(r * CAP + lo, 8)

            def meta_start(ci, pkb, cfb):
                pltpu.async_copy(
                    pk_hbm.at[pl.ds(rbase + ci * CHUNK, CHUNK)], pkb, semM)
                pltpu.async_copy(
                    cf_hbm.at[pl.ds(rbase + ci * CHUNK, CHUNK)],
                    cfb.at[pl.ds(0, CHUNK)], semM)

            def meta_wait(ci, pkb, cfb):
                pltpu.make_async_copy(
                    pk_hbm.at[pl.ds(rbase + ci * CHUNK, CHUNK)], pkb,
                    semM).wait()
                pltpu.make_async_copy(
                    cf_hbm.at[pl.ds(rbase + ci * CHUNK, CHUNK)],
                    cfb.at[pl.ds(0, CHUNK)], semM).wait()

            def build(pkb, idxb, dstlb):
                for j in range(CHUNK // 16):
                    v = pkb[pl.ds(j * 16, 16)]
                    idxb[pl.ds(j * 16, 16)] = lax.shift_right_logical(v, 7)
                    dstlb[pl.ds(j * 16, 16)] = lax.bitwise_and(v, BROWS - 1)

            def gather_start(idxb, rowsb, semX):
                @pl.when(r == 0)
                def _():
                    pltpu.async_copy(t0_hbm.at[idxb], rowsb, semX)

                @pl.when(r == 1)
                def _():
                    pltpu.async_copy(t1_hbm.at[idxb], rowsb, semX)

            def gather_wait(idxb, rowsb, semX):
                @pl.when(r == 0)
                def _():
                    pltpu.make_async_copy(t0_hbm.at[idxb], rowsb, semX).wait()

                @pl.when(r == 1)
                def _():
                    pltpu.make_async_copy(t1_hbm.at[idxb], rowsb, semX).wait()

            def compute(rowsb, cfb, dstlb):
                @plsc.parallel_loop(0, CHUNK, 1, unroll=4)
                def _(i):
                    cs = cfb[pl.ds(i, 16)][0]
                    dbase = dstlb[pl.ds(i, 16)][0] * D
                    for j in range(D // 16):
                        v = rowsb[pl.ds(i, 1), pl.ds(j * 16, 16)]
                        plsc.addupdate(
                            acc.at[pl.ds(dbase + j * 16, 16)],
                            (cs * v).reshape(16))

            @pl.when(nch > 0)
            def _():
                meta_start(0, pk0b, cf0b)
                meta_wait(0, pk0b, cf0b)
                build(pk0b, idx0, dstl0)
                gather_start(idx0, rows0, semA0)

                @pl.when(nch > 1)
                def _():
                    meta_start(1, pk1b, cf1b)

            def stage_body(ci, pkb_n, cfb_n, idxb_n, dstlb_n, rowsb_n, semN,
                           pkb_c, cfb_c, idxb_c, dstlb_c, rowsb_c, semC):
                @pl.when(ci + 1 < nch)
                def _():
                    meta_wait(ci + 1, pkb_n, cfb_n)
                    build(pkb_n, idxb_n, dstlb_n)
                    gather_start(idxb_n, rowsb_n, semN)

                gather_wait(idxb_c, rowsb_c, semC)
                compute(rowsb_c, cfb_c, dstlb_c)

                @pl.when(ci + 2 < nch)
                def _():
                    meta_start(ci + 2, pkb_c, cfb_c)

            @pl.loop(0, nch)
            def _(ci):
                @pl.when(lax.rem(ci, 2) == 0)
                def _():
                    stage_body(ci, pk1b, cf1b, idx1, dstl1, rows1, semA1,
                               pk0b, cf0b, idx0, dstl0, rows0, semA0)

                @pl.when(lax.rem(ci, 2) == 1)
                def _():
                    stage_body(ci, pk0b, cf0b, idx0, dstl0, rows0, semA0,
                               pk1b, cf1b, idx1, dstl1, rows1, semA1)

            row0 = (r * N + b * BROWS) * D

            @pl.when(b < NBUCK - 1)
            def _():
                pltpu.sync_copy(acc, p_hbm.at[pl.ds(row0, BROWS * D)])

            @pl.when(b == NBUCK - 1)
            def _():
                pltpu.sync_copy(acc.at[pl.ds(0, nlast * D)],
                                p_hbm.at[pl.ds(row0, nlast * D)])


_sc_prop = pl.kernel(
    _sc_prop_body,
    mesh=plsc.VectorSubcoreMesh(core_axis_name="c", subcore_axis_name="s"),
    out_type=jax.ShapeDtypeStruct((2 * N * D,), jnp.float32),
    scratch_types=[
        pltpu.VMEM((BROWS * D,), jnp.float32),
        pltpu.VMEM((CHUNK, D), jnp.float32),
        pltpu.VMEM((CHUNK, D), jnp.float32),
        pltpu.VMEM((CHUNK,), jnp.int32),
        pltpu.VMEM((CHUNK,), jnp.int32),
        pltpu.VMEM((CHUNK + 16,), jnp.float32),
        pltpu.VMEM((CHUNK + 16,), jnp.float32),
        pltpu.VMEM((CHUNK,), jnp.int32),
        pltpu.VMEM((CHUNK,), jnp.int32),
        pltpu.VMEM((CHUNK + 16,), jnp.int32),
        pltpu.VMEM((CHUNK + 16,), jnp.int32),
        pltpu.VMEM((2 * (NBUCK + 1) + 16,), jnp.int32),
        pltpu.SemaphoreType.DMA,
        pltpu.SemaphoreType.DMA,
        pltpu.SemaphoreType.DMA,
    ],
)


def _quake_rsqrt(v):
    i = plsc.bitcast(v, jnp.int32)
    i = 0x5F3759DF - lax.shift_right_logical(i, 1)
    y = plsc.bitcast(i, jnp.float32)
    for _ in range(3):
        y = y * (1.5 - 0.5 * v * y * y)
    return y


ELAST = E - 15 * PER
ILAST = CAP - 15 * PINIT
RLAST = NHIST - 15 * RSL


def _sc_prep_body(s_hbm, d_hbm, pk_hbm, cf_hbm, off_hbm,
                  sv, dv, hs, hd, rbuf, cbuf, gv, lc, offv, msv,
                  cev, pkv, posv, pb0, pb1, ibuf, zbuf,
                  stage_s, stage_d, comb_s, comb_d, grid_spm,
                  cnt_s, sem, semp):
    c = lax.axis_index("c")
    t = lax.axis_index("s")
    ones16 = jnp.full((16,), 1.0, jnp.float32)
    z16f = jnp.zeros((16,), jnp.float32)
    iota16 = lax.iota(jnp.int32, 16)

    pltpu.sync_copy(s_hbm.at[pl.ds(c * E + t * PER, PER)], sv)
    pltpu.sync_copy(d_hbm.at[pl.ds(c * E + t * PER, PER)], dv)

    @pl.when(t == 15)
    def _():
        for j in range((PER - ELAST) // 16):
            sv[pl.ds(ELAST + 16 * j, 16)] = jnp.full((16,), SENT_NODE, jnp.int32)
            dv[pl.ds(ELAST + 16 * j, 16)] = jnp.full((16,), SENT_DST, jnp.int32)

    @plsc.parallel_loop(0, NHIST // 16, 1, unroll=4)
    def _(k):
        hs[pl.ds(k * 16, 16)] = z16f
        hd[pl.ds(k * 16, 16)] = z16f

    for k in range(6):
        lc[pl.ds(k * 16, 16)] = z16f

    @plsc.parallel_loop(0, NGRP, 1, unroll=4)
    def _(g):
        svv = sv[pl.ds(g * 16, 16)]
        dvv = dv[pl.ds(g * 16, 16)]
        plsc.addupdate_scatter(hs, [svv], ones16)
        plsc.addupdate_scatter(hd, [dvv], ones16)
        plsc.addupdate_scatter(lc, [lax.shift_right_logical(dvv, 7)], ones16)

    pltpu.sync_copy(hs, stage_s.at[pl.ds(t * NHIST, NHIST)])
    pltpu.sync_copy(hd, stage_d.at[pl.ds(t * NHIST, NHIST)])
    pltpu.sync_copy(lc.at[pl.ds(0, 80)], grid_spm.at[pl.ds(t * 80, 80)])
    plsc.subcore_barrier()

    def _reduce(stage, comb):
        for tt in range(16):
            pltpu.async_copy(stage.at[pl.ds(tt * NHIST + t * RSL, RSL)],
                             rbuf.at[pl.ds(tt * RSL, RSL)], sem)
        for tt in range(16):
            pltpu.make_async_copy(stage.at[pl.ds(tt * NHIST + t * RSL, RSL)],
                                  rbuf.at[pl.ds(tt * RSL, RSL)], sem).wait()

        @plsc.parallel_loop(0, RSL // 16, 1, unroll=2)
        def _(k):
            s = rbuf[pl.ds(k * 16, 16)]
            for tt in range(1, 16):
                s = s + rbuf[pl.ds(tt * RSL + k * 16, 16)]
            cbuf[pl.ds(k * 16, 16)] = s

        @pl.when(t < 15)
        def _():
            pltpu.sync_copy(cbuf, comb.at[pl.ds(t * RSL, RSL)])

        @pl.when(t == 15)
        def _():
            pltpu.sync_copy(cbuf.at[pl.ds(0, RLAST)],
                            comb.at[pl.ds(t * RSL, RLAST)])

    _reduce(stage_s, comb_s)
    _reduce(stage_d, comb_d)
    plsc.subcore_barrier()

    pltpu.sync_copy(comb_s, hs)
    pltpu.sync_copy(comb_d, hd)

    @plsc.parallel_loop(0, NHIST // 16, 1, unroll=4)
    def _(k):
        hs[pl.ds(k * 16, 16)] = _quake_rsqrt(
            jnp.maximum(hs[pl.ds(k * 16, 16)], 1.0))
        hd[pl.ds(k * 16, 16)] = _quake_rsqrt(
            jnp.maximum(hd[pl.ds(k * 16, 16)], 1.0))

    @plsc.parallel_loop(0, NGRP, 1, unroll=4)
    def _(g):
        svv = sv[pl.ds(g * 16, 16)]
        dvv = dv[pl.ds(g * 16, 16)]
        cev[pl.ds(g * 16, 16)] = (plsc.load_gather(hs, [svv]) *
                                  plsc.load_gather(hd, [dvv]))

    pltpu.sync_copy(grid_spm, gv.at[pl.ds(0, 1280)])
    carry = jnp.int32(0)
    for k in range(5):
        s = gv[pl.ds(k * 16, 16)]
        for tt in range(1, 16):
            s = s + gv[pl.ds(tt * 80 + k * 16, 16)]
        cnt_i = s.astype(jnp.int32)
        pad = ((cnt_i + (CHUNK - 1)) // CHUNK) * CHUNK
        if k == 4:
            pad = jnp.where(iota16 == 15, 0, pad)
        incl = plsc.cumsum(pad) + carry
        offv[pl.ds(k * 16, 16)] = incl - pad
        carry = incl[15]

    @pl.when(t == 0)
    def _():
        pltpu.sync_copy(offv.at[pl.ds(0, 80)], off_hbm.at[pl.ds(c * 80, 80)])

    for k in range(5):
        msv[pl.ds(k * 16, 16)] = offv[pl.ds(k * 16, 16)] + c * CAP

    @pl.loop(0, t)
    def _(tt):
        for k in range(5):
            msv[pl.ds(k * 16, 16)] = (msv[pl.ds(k * 16, 16)] +
                                      gv[pl.ds(tt * 80 + k * 16, 16)].astype(jnp.int32))

    for k in range(5):
        v = msv[pl.ds(k * 16, 16)]
        for lane in range(16):
            cnt_s[k * 16 + lane] = v[lane]

    @plsc.parallel_loop(0, PINIT // 16, 1, unroll=4)
    def _(j):
        g = t * PINIT + j * 16 + iota16
        ibuf[pl.ds(j * 16, 16)] = lax.rem(g, N) * BROWS
        zbuf[pl.ds(j * 16, 16)] = z16f

    @pl.when(t < 15)
    def _():
        pltpu.sync_copy(ibuf, pk_hbm.at[pl.ds(c * CAP + t * PINIT, PINIT)])
        pltpu.sync_copy(zbuf, cf_hbm.at[pl.ds(c * CAP + t * PINIT, PINIT)])

    @pl.when(t == 15)
    def _():
        pltpu.sync_copy(ibuf.at[pl.ds(0, ILAST)],
                        pk_hbm.at[pl.ds(c * CAP + 15 * PINIT, ILAST)])
        pltpu.sync_copy(zbuf.at[pl.ds(0, ILAST)],
                        cf_hbm.at[pl.ds(c * CAP + 15 * PINIT, ILAST)])

    plsc.subcore_barrier()

    @plsc.parallel_loop(0, NPOS // 16, 1, unroll=4)
    def _(j):
        posv[pl.ds(j * 16, 16)] = (c * CAP + DATA_CAP +
                                   lax.rem(j * 16 + iota16, 256))

    @pl.loop(0, NGRP)
    def _(g):
        dvv = dv[pl.ds(g * 16, 16)]
        bv = lax.shift_right_logical(dvv, 7)
        pv = jnp.zeros((16,), jnp.int32)
        for lane in range(16):
            b = bv[lane]
            p = cnt_s[b]
            cnt_s[b] = p + 1
            pv = jnp.where(iota16 == lane, p, pv)
        posv[pl.ds(g * 16, 16)] = pv
        pkv[pl.ds(g * 16, 16)] = (sv[pl.ds(g * 16, 16)] * BROWS +
                                  lax.bitwise_and(dvv, BROWS - 1))

    nsc = NPOS // 128
    for j in range(nsc):
        pb = pb0 if j % 2 == 0 else pb1
        if j >= 2:
            pltpu.make_async_copy(pkv.at[pl.ds((j - 2) * 128, 128)],
                                  pk_hbm.at[pb], semp).wait()
            pltpu.make_async_copy(cev.at[pl.ds((j - 2) * 128, 128)],
                                  cf_hbm.at[pb], semp).wait()
        for k in range(8):
            pb[pl.ds(k * 16, 16)] = posv[pl.ds(j * 128 + k * 16, 16)]
        pltpu.async_copy(pkv.at[pl.ds(j * 128, 128)], pk_hbm.at[pb], semp)
        pltpu.async_copy(cev.at[pl.ds(j * 128, 128)], cf_hbm.at[pb], semp)
    for j in range(nsc - 2, nsc):
        pb = pb0 if j % 2 == 0 else pb1
        pltpu.make_async_copy(pkv.at[pl.ds(j * 128, 128)],
                              pk_hbm.at[pb], semp).wait()
        pltpu.make_async_copy(cev.at[pl.ds(j * 128, 128)],
                              cf_hbm.at[pb], semp).wait()


_sc_cp = pltpu.CompilerParams()
if "needs_layout_passes" in pltpu.CompilerParams.__dataclass_fields__:
    _sc_cp = dataclasses.replace(_sc_cp, needs_layout_passes=False)

_sc_prep = pl.kernel(
    _sc_prep_body,
    mesh=plsc.VectorSubcoreMesh(core_axis_name="c", subcore_axis_name="s"),
    compiler_params=_sc_cp,
    out_type=[
        jax.ShapeDtypeStruct((2 * CAP,), jnp.int32),
        jax.ShapeDtypeStruct((2 * CAP,), jnp.float32),
        jax.ShapeDtypeStruct((2 * (NBUCK + 1),), jnp.int32),
    ],
    scratch_types=[
        pltpu.VMEM((PER,), jnp.int32),
        pltpu.VMEM((PER,), jnp.int32),
        pltpu.VMEM((NHIST,), jnp.float32),
        pltpu.VMEM((NHIST,), jnp.float32),
        pltpu.VMEM((16 * RSL,), jnp.float32),
        pltpu.VMEM((RSL,), jnp.float32),
        pltpu.VMEM((1280 + 16,), jnp.float32),
        pltpu.VMEM((96,), jnp.float32),
        pltpu.VMEM((96,), jnp.int32),
        pltpu.VMEM((96,), jnp.int32),
        pltpu.VMEM((NPOS,), jnp.float32),
        pltpu.VMEM((NPOS,), jnp.int32),
        pltpu.VMEM((NPOS,), jnp.int32),
        pltpu.VMEM((128,), jnp.int32),
        pltpu.VMEM((128,), jnp.int32),
        pltpu.VMEM((PINIT,), jnp.int32),
        pltpu.VMEM((PINIT,), jnp.float32),
        pltpu.VMEM_SHARED((16 * NHIST + 112,), jnp.float32),
        pltpu.VMEM_SHARED((16 * NHIST + 112,), jnp.float32),
        pltpu.VMEM_SHARED((NHIST,), jnp.float32),
        pltpu.VMEM_SHARED((NHIST,), jnp.float32),
        pltpu.VMEM_SHARED((1280,), jnp.float32),
        pltpu.SMEM((96,), jnp.int32),
        pltpu.SemaphoreType.DMA,
        pltpu.SemaphoreType.DMA,
    ],
)


def _prep_jax(ei):
    src, dst = ei[0], ei[1]
    E = src.shape[0]
    ones = jnp.ones((E,), jnp.float32)
    deg_out = jax.ops.segment_sum(ones, src, num_segments=N)
    deg_in = jax.ops.segment_sum(ones, dst, num_segments=N)
    ns = jax.lax.rsqrt(jnp.maximum(deg_out, 1.0))
    nd = jax.lax.rsqrt(jnp.maximum(deg_in, 1.0))
    coef = ns[src] * nd[dst]
    bucket = dst // BROWS
    order = jnp.argsort(bucket, stable=True)
    srcs = src[order]
    dsts = dst[order]
    coefs = coef[order]
    cnt = jax.ops.segment_sum(jnp.ones((E,), jnp.int32), bucket,
                              num_segments=NBUCK)
    padded = ((cnt + CHUNK - 1) // CHUNK) * CHUNK
    off = jnp.concatenate(
        [jnp.zeros((1,), jnp.int32), jnp.cumsum(padded, dtype=jnp.int32)])
    start = jnp.cumsum(cnt) - cnt
    bsort = bucket[order]
    rank = jnp.arange(E, dtype=jnp.int32) - start[bsort]
    pos = off[bsort] + rank
    packed = (jnp.arange(CAP, dtype=jnp.int32) % N) * BROWS
    packed = packed.at[pos].set(srcs * BROWS + (dsts % BROWS))
    coefa = jnp.zeros((CAP,), jnp.float32).at[pos].set(coefs)
    return packed, coefa, off


def kernel(x, edge_index_rel0, edge_index_rel1, W00, b00, W01, b01, W10, b10, W11, b11):
    b00 = b00.reshape(1, D)
    b01 = b01.reshape(1, D)
    b10 = b10.reshape(1, D)
    b11 = b11.reshape(1, D)
    zpad = jnp.zeros((80,), jnp.int32)
    s_all = jnp.concatenate([edge_index_rel0[0], edge_index_rel1[0], zpad])
    d_all = jnp.concatenate([edge_index_rel0[1], edge_index_rel1[1], zpad])
    packed, coef, off = _sc_prep(s_all, d_all)
    t0, t1 = _tc_mm2(x, W00, W01)
    p_a = _sc_prop(t0, t1, packed, coef, off).reshape(2 * N, D)
    t2, t3 = _tc_mid(p_a, p_a, b00, b01, W10, W11)
    p_b = _sc_prop(t2, t3, packed, coef, off).reshape(2 * N, D)
    return _tc_fin(p_b, p_b, b10, b11, x)


# speedup vs baseline: 1.7306x; 1.0491x over previous
import dataclasses
import functools

import jax
import jax.numpy as jnp
from jax import lax
from jax.experimental import pallas as pl
from jax.experimental.pallas import tpu as pltpu
from jax.experimental.pallas import tpu_sc as plsc

N = 10000
D = 512
BLK = 400

BROWS = 128
NBUCK = (N + BROWS - 1) // BROWS
NITEM = 2 * NBUCK
CHUNK = 48
E = 78000
DATA_CAP = 81696
CAP = DATA_CAP + 256
PER = 4880
NGRP = PER // 16
NHIST = 10128
SENT_NODE = 10008
SENT_DST = 10112
RSL = 640
PINIT = 5136
NPOS = 4992


def _mm2_body(x_ref, w0_ref, w1_ref, t0_ref, t1_ref):
    xb = x_ref[...]
    t0_ref[...] = jnp.dot(xb, w0_ref[...], preferred_element_type=jnp.float32)
    t1_ref[...] = jnp.dot(xb, w1_ref[...], preferred_element_type=jnp.float32)


def _tc_mm2(x, w0, w1):
    grid = (N // BLK,)
    return pl.pallas_call(
        _mm2_body,
        grid=grid,
        in_specs=[
            pl.BlockSpec((BLK, D), lambda i: (i, 0)),
            pl.BlockSpec((D, D), lambda i: (0, 0)),
            pl.BlockSpec((D, D), lambda i: (0, 0)),
        ],
        out_specs=[
            pl.BlockSpec((BLK, D), lambda i: (i, 0)),
            pl.BlockSpec((BLK, D), lambda i: (i, 0)),
        ],
        out_shape=[
            jax.ShapeDtypeStruct((N, D), jnp.float32),
            jax.ShapeDtypeStruct((N, D), jnp.float32),
        ],
    )(x, w0, w1)


def _mid_body(p0_ref, p1_ref, b0_ref, b1_ref, w0_ref, w1_ref, t0_ref, t1_ref):
    h = jax.nn.relu(p0_ref[...] + p1_ref[...] + b0_ref[...] + b1_ref[...])
    t0_ref[...] = jnp.dot(h, w0_ref[...], preferred_element_type=jnp.float32)
    t1_ref[...] = jnp.dot(h, w1_ref[...], preferred_element_type=jnp.float32)


def _tc_mid(p0, p1, b0, b1, w0, w1):
    grid = (N // BLK,)
    return pl.pallas_call(
        _mid_body,
        grid=grid,
        in_specs=[
            pl.BlockSpec((BLK, D), lambda i: (i, 0)),
            pl.BlockSpec((BLK, D), lambda i: (i + N // BLK, 0)),
            pl.BlockSpec((1, D), lambda i: (0, 0)),
            pl.BlockSpec((1, D), lambda i: (0, 0)),
            pl.BlockSpec((D, D), lambda i: (0, 0)),
            pl.BlockSpec((D, D), lambda i: (0, 0)),
        ],
        out_specs=[
            pl.BlockSpec((BLK, D), lambda i: (i, 0)),
            pl.BlockSpec((BLK, D), lambda i: (i, 0)),
        ],
        out_shape=[
            jax.ShapeDtypeStruct((N, D), jnp.float32),
            jax.ShapeDtypeStruct((N, D), jnp.float32),
        ],
    )(p0, p1, b0, b1, w0, w1)


def _fin_body(p0_ref, p1_ref, b0_ref, b1_ref, x_ref, o_ref):
    h = jax.nn.relu(p0_ref[...] + p1_ref[...] + b0_ref[...] + b1_ref[...])
    o_ref[...] = jax.nn.relu(h + x_ref[...])


def _tc_fin(p0, p1, b0, b1, x):
    grid = (N // BLK,)
    return pl.pallas_call(
        _fin_body,
        grid=grid,
        in_specs=[
            pl.BlockSpec((BLK, D), lambda i: (i, 0)),
            pl.BlockSpec((BLK, D), lambda i: (i + N // BLK, 0)),
            pl.BlockSpec((1, D), lambda i: (0, 0)),
            pl.BlockSpec((1, D), lambda i: (0, 0)),
            pl.BlockSpec((BLK, D), lambda i: (i, 0)),
        ],
        out_specs=pl.BlockSpec((BLK, D), lambda i: (i, 0)),
        out_shape=jax.ShapeDtypeStruct((N, D), jnp.float32),
    )(p0, p1, b0, b1, x)


def _sc_prop_body(t0_hbm, t1_hbm, pk_hbm, cf_hbm, off_hbm, p_hbm,
                  acc, rows0, rows1, pk0b, pk1b, cf0b, cf1b,
                  idx0, idx1, dstl0, dstl1, offs_v, semA0, semA1, semM):
    wid = lax.axis_index("c") * 16 + lax.axis_index("s")
    pltpu.sync_copy(off_hbm, offs_v.at[pl.ds(0, 2 * (NBUCK + 1))])
    nlast = N - (NBUCK - 1) * BROWS

    @pl.loop(0, (NITEM + 31) // 32)
    def _(k):
        item = wid + 32 * k

        @pl.when(item < NITEM)
        def _():
            r = item // NBUCK
            b = item - r * NBUCK
            zero = jnp.zeros((16,), jnp.float32)

            @plsc.parallel_loop(0, BROWS * D // 256, 1, unroll=2)
            def _(rr):
                for j in range(16):
                    acc[pl.ds(rr * 256 + j * 16, 16)] = zero

            ovec = offs_v[pl.ds(r * (NBUCK + 1) + b, 16)]
            lo = pl.multiple_of(ovec[0], CHUNK)
            hi = ovec[1]
            nch = (hi - lo) // CHUNK
            rbase = pl.multiple_of(r * CAP + lo, 8)

            def meta_start(ci, pkb, cfb):
                pltpu.async_copy(
                    pk_hbm.at[pl.ds(rbase + ci * CHUNK, CHUNK)], pkb, semM)
                pltpu.async_copy(
                    cf_hbm.at[pl.ds(rbase + ci * CHUNK, CHUNK)],
                    cfb.at[pl.ds(0, CHUNK)], semM)

            def meta_wait(ci, pkb, cfb):
                pltpu.make_async_copy(
                    pk_hbm.at[pl.ds(rbase + ci * CHUNK, CHUNK)], pkb,
                    semM).wait()
                pltpu.make_async_copy(
                    cf_hbm.at[pl.ds(rbase + ci * CHUNK, CHUNK)],
                    cfb.at[pl.ds(0, CHUNK)], semM).wait()

            def build(pkb, idxb, dstlb):
                for j in range(CHUNK // 16):
                    v = pkb[pl.ds(j * 16, 16)]
                    idxb[pl.ds(j * 16, 16)] = lax.shift_right_logical(v, 7)
                    dstlb[pl.ds(j * 16, 16)] = lax.bitwise_and(v, BROWS - 1)

            def gather_start(idxb, rowsb, semX):
                @pl.when(r == 0)
                def _():
                    pltpu.async_copy(t0_hbm.at[idxb], rowsb, semX)

                @pl.when(r == 1)
                def _():
                    pltpu.async_copy(t1_hbm.at[idxb], rowsb, semX)

            def gather_wait(idxb, rowsb, semX):
                @pl.when(r == 0)
                def _():
                    pltpu.make_async_copy(t0_hbm.at[idxb], rowsb, semX).wait()

                @pl.when(r == 1)
                def _():
                    pltpu.make_async_copy(t1_hbm.at[idxb], rowsb, semX).wait()

            def compute(rowsb, cfb, dstlb):
                @plsc.parallel_loop(0, CHUNK, 1, unroll=8)
                def _(i):
                    cs = cfb[pl.ds(i, 16)][0]
                    dbase = dstlb[pl.ds(i, 16)][0] * D
                    for j in range(D // 16):
                        v = rowsb[pl.ds(i, 1), pl.ds(j * 16, 16)]
                        plsc.addupdate(
                            acc.at[pl.ds(dbase + j * 16, 16)],
                            (cs * v).reshape(16))

            @pl.when(nch > 0)
            def _():
                meta_start(0, pk0b, cf0b)
                meta_wait(0, pk0b, cf0b)
                build(pk0b, idx0, dstl0)
                gather_start(idx0, rows0, semA0)

                @pl.when(nch > 1)
                def _():
                    meta_start(1, pk1b, cf1b)

            def stage_body(ci, pkb_n, cfb_n, idxb_n, dstlb_n, rowsb_n, semN,
                           pkb_c, cfb_c, idxb_c, dstlb_c, rowsb_c, semC):
                @pl.when(ci + 1 < nch)
                def _():
                    meta_wait(ci + 1, pkb_n, cfb_n)
                    build(pkb_n, idxb_n, dstlb_n)
                    gather_start(idxb_n, rowsb_n, semN)

                gather_wait(idxb_c, rowsb_c, semC)
                compute(rowsb_c, cfb_c, dstlb_c)

                @pl.when(ci + 2 < nch)
                def _():
                    meta_start(ci + 2, pkb_c, cfb_c)

            @pl.loop(0, nch)
            def _(ci):
                @pl.when(lax.rem(ci, 2) == 0)
                def _():
                    stage_body(ci, pk1b, cf1b, idx1, dstl1, rows1, semA1,
                               pk0b, cf0b, idx0, dstl0, rows0, semA0)

                @pl.when(lax.rem(ci, 2) == 1)
                def _():
                    stage_body(ci, pk0b, cf0b, idx0, dstl0, rows0, semA0,
                               pk1b, cf1b, idx1, dstl1, rows1, semA1)

            row0 = (r * N + b * BROWS) * D

            @pl.when(b < NBUCK - 1)
            def _():
                pltpu.sync_copy(acc, p_hbm.at[pl.ds(row0, BROWS * D)])

            @pl.when(b == NBUCK - 1)
            def _():
                pltpu.sync_copy(acc.at[pl.ds(0, nlast * D)],
                                p_hbm.at[pl.ds(row0, nlast * D)])


_sc_prop = pl.kernel(
    _sc_prop_body,
    mesh=plsc.VectorSubcoreMesh(core_axis_name="c", subcore_axis_name="s"),
    out_type=jax.ShapeDtypeStruct((2 * N * D,), jnp.float32),
    scratch_types=[
        pltpu.VMEM((BROWS * D,), jnp.float32),
        pltpu.VMEM((CHUNK, D), jnp.float32),
        pltpu.VMEM((CHUNK, D), jnp.float32),
        pltpu.VMEM((CHUNK,), jnp.int32),
        pltpu.VMEM((CHUNK,), jnp.int32),
        pltpu.VMEM((CHUNK + 16,), jnp.float32),
        pltpu.VMEM((CHUNK + 16,), jnp.float32),
        pltpu.VMEM((CHUNK,), jnp.int32),
        pltpu.VMEM((CHUNK,), jnp.int32),
        pltpu.VMEM((CHUNK + 16,), jnp.int32),
        pltpu.VMEM((CHUNK + 16,), jnp.int32),
        pltpu.VMEM((2 * (NBUCK + 1) + 16,), jnp.int32),
        pltpu.SemaphoreType.DMA,
        pltpu.SemaphoreType.DMA,
        pltpu.SemaphoreType.DMA,
    ],
)


def _quake_rsqrt(v):
    i = plsc.bitcast(v, jnp.int32)
    i = 0x5F3759DF - lax.shift_right_logical(i, 1)
    y = plsc.bitcast(i, jnp.float32)
    for _ in range(3):
        y = y * (1.5 - 0.5 * v * y * y)
    return y


ELAST = E - 15 * PER
ILAST = CAP - 15 * PINIT
RLAST = NHIST - 15 * RSL


def _sc_prep_body(s_hbm, d_hbm, pk_hbm, cf_hbm, off_hbm,
                  sv, dv, hs, hd, rbuf, cbuf, gv, lc, offv, msv,
                  cev, pkv, posv, pb0, pb1, pb2, pb3, ibuf, zbuf,
                  stage_s, stage_d, comb_s, comb_d, grid_spm,
                  cnt_s, sem, semp):
    c = lax.axis_index("c")
    t = lax.axis_index("s")
    ones16 = jnp.full((16,), 1.0, jnp.float32)
    z16f = jnp.zeros((16,), jnp.float32)
    iota16 = lax.iota(jnp.int32, 16)

    pltpu.sync_copy(s_hbm.at[pl.ds(c * E + t * PER, PER)], sv)
    pltpu.sync_copy(d_hbm.at[pl.ds(c * E + t * PER, PER)], dv)

    @pl.when(t == 15)
    def _():
        for j in range((PER - ELAST) // 16):
            sv[pl.ds(ELAST + 16 * j, 16)] = jnp.full((16,), SENT_NODE, jnp.int32)
            dv[pl.ds(ELAST + 16 * j, 16)] = jnp.full((16,), SENT_DST, jnp.int32)

    @plsc.parallel_loop(0, NHIST // 16, 1, unroll=4)
    def _(k):
        hs[pl.ds(k * 16, 16)] = z16f
        hd[pl.ds(k * 16, 16)] = z16f

    for k in range(6):
        lc[pl.ds(k * 16, 16)] = z16f

    @plsc.parallel_loop(0, NGRP, 1, unroll=4)
    def _(g):
        svv = sv[pl.ds(g * 16, 16)]
        dvv = dv[pl.ds(g * 16, 16)]
        plsc.addupdate_scatter(hs, [svv], ones16)
        plsc.addupdate_scatter(hd, [dvv], ones16)
        plsc.addupdate_scatter(lc, [lax.shift_right_logical(dvv, 7)], ones16)

    pltpu.sync_copy(hs, stage_s.at[pl.ds(t * NHIST, NHIST)])
    pltpu.sync_copy(hd, stage_d.at[pl.ds(t * NHIST, NHIST)])
    pltpu.sync_copy(lc.at[pl.ds(0, 80)], grid_spm.at[pl.ds(t * 80, 80)])
    plsc.subcore_barrier()

    def _reduce(stage, comb):
        for tt in range(16):
            pltpu.async_copy(stage.at[pl.ds(tt * NHIST + t * RSL, RSL)],
                             rbuf.at[pl.ds(tt * RSL, RSL)], sem)
        for tt in range(16):
            pltpu.make_async_copy(stage.at[pl.ds(tt * NHIST + t * RSL, RSL)],
                                  rbuf.at[pl.ds(tt * RSL, RSL)], sem).wait()

        @plsc.parallel_loop(0, RSL // 16, 1, unroll=2)
        def _(k):
            s = rbuf[pl.ds(k * 16, 16)]
            for tt in range(1, 16):
                s = s + rbuf[pl.ds(tt * RSL + k * 16, 16)]
            cbuf[pl.ds(k * 16, 16)] = s

        @pl.when(t < 15)
        def _():
            pltpu.sync_copy(cbuf, comb.at[pl.ds(t * RSL, RSL)])

        @pl.when(t == 15)
        def _():
            pltpu.sync_copy(cbuf.at[pl.ds(0, RLAST)],
                            comb.at[pl.ds(t * RSL, RLAST)])

    _reduce(stage_s, comb_s)
    _reduce(stage_d, comb_d)
    plsc.subcore_barrier()

    pltpu.sync_copy(comb_s, hs)
    pltpu.sync_copy(comb_d, hd)

    @plsc.parallel_loop(0, NHIST // 16, 1, unroll=4)
    def _(k):
        hs[pl.ds(k * 16, 16)] = _quake_rsqrt(
            jnp.maximum(hs[pl.ds(k * 16, 16)], 1.0))
        hd[pl.ds(k * 16, 16)] = _quake_rsqrt(
            jnp.maximum(hd[pl.ds(k * 16, 16)], 1.0))

    @plsc.parallel_loop(0, NGRP, 1, unroll=4)
    def _(g):
        svv = sv[pl.ds(g * 16, 16)]
        dvv = dv[pl.ds(g * 16, 16)]
        cev[pl.ds(g * 16, 16)] = (plsc.load_gather(hs, [svv]) *
                                  plsc.load_gather(hd, [dvv]))

    pltpu.sync_copy(grid_spm, gv.at[pl.ds(0, 1280)])
    carry = jnp.int32(0)
    for k in range(5):
        s = gv[pl.ds(k * 16, 16)]
        for tt in range(1, 16):
            s = s + gv[pl.ds(tt * 80 + k * 16, 16)]
        cnt_i = s.astype(jnp.int32)
        pad = ((cnt_i + (CHUNK - 1)) // CHUNK) * CHUNK
        if k == 4:
            pad = jnp.where(iota16 == 15, 0, pad)
        incl = plsc.cumsum(pad) + carry
        offv[pl.ds(k * 16, 16)] = incl - pad
        carry = incl[15]

    @pl.when(t == 0)
    def _():
        pltpu.sync_copy(offv.at[pl.ds(0, 80)], off_hbm.at[pl.ds(c * 80, 80)])

    for k in range(5):
        msv[pl.ds(k * 16, 16)] = offv[pl.ds(k * 16, 16)] + c * CAP

    @pl.loop(0, t)
    def _(tt):
        for k in range(5):
            msv[pl.ds(k * 16, 16)] = (msv[pl.ds(k * 16, 16)] +
                                      gv[pl.ds(tt * 80 + k * 16, 16)].astype(jnp.int32))

    for k in range(5):
        v = msv[pl.ds(k * 16, 16)]
        for lane in range(16):
            cnt_s[k * 16 + lane] = v[lane]

    @plsc.parallel_loop(0, PINIT // 16, 1, unroll=4)
    def _(j):
        g = t * PINIT + j * 16 + iota16
        ibuf[pl.ds(j * 16, 16)] = lax.rem(g, N) * BROWS
        zbuf[pl.ds(j * 16, 16)] = z16f

    @pl.when(t < 15)
    def _():
        pltpu.sync_copy(ibuf, pk_hbm.at[pl.ds(c * CAP + t * PINIT, PINIT)])
        pltpu.sync_copy(zbuf, cf_hbm.at[pl.ds(c * CAP + t * PINIT, PINIT)])

    @pl.when(t == 15)
    def _():
        pltpu.sync_copy(ibuf.at[pl.ds(0, ILAST)],
                        pk_hbm.at[pl.ds(c * CAP + 15 * PINIT, ILAST)])
        pltpu.sync_copy(zbuf.at[pl.ds(0, ILAST)],
                        cf_hbm.at[pl.ds(c * CAP + 15 * PINIT, ILAST)])

    plsc.subcore_barrier()

    @plsc.parallel_loop(0, NPOS // 16, 1, unroll=4)
    def _(j):
        posv[pl.ds(j * 16, 16)] = (c * CAP + DATA_CAP +
                                   lax.rem(j * 16 + iota16, 256))

    @pl.loop(0, NGRP)
    def _(g):
        dvv = dv[pl.ds(g * 16, 16)]
        bv = lax.shift_right_logical(dvv, 7)
        pv = jnp.zeros((16,), jnp.int32)
        for lane in range(16):
            b = bv[lane]
            p = cnt_s[b]
            cnt_s[b] = p + 1
            pv = jnp.where(iota16 == lane, p, pv)
        posv[pl.ds(g * 16, 16)] = pv
        pkv[pl.ds(g * 16, 16)] = (sv[pl.ds(g * 16, 16)] * BROWS +
                                  lax.bitwise_and(dvv, BROWS - 1))

    nsc = NPOS // 128
    pbs = [pb0, pb1, pb2, pb3]
    nd = len(pbs)
    for j in range(nsc):
        pb = pbs[j % nd]
        if j >= nd:
            pltpu.make_async_copy(pkv.at[pl.ds((j - nd) * 128, 128)],
                                  pk_hbm.at[pb], semp).wait()
            pltpu.make_async_copy(cev.at[pl.ds((j - nd) * 128, 128)],
                                  cf_hbm.at[pb], semp).wait()
        for k in range(8):
            pb[pl.ds(k * 16, 16)] = posv[pl.ds(j * 128 + k * 16, 16)]
        pltpu.async_copy(pkv.at[pl.ds(j * 128, 128)], pk_hbm.at[pb], semp)
        pltpu.async_copy(cev.at[pl.ds(j * 128, 128)], cf_hbm.at[pb], semp)
    for j in range(nsc - nd, nsc):
        pb = pbs[j % nd]
        pltpu.make_async_copy(pkv.at[pl.ds(j * 128, 128)],
                              pk_hbm.at[pb], semp).wait()
        pltpu.make_async_copy(cev.at[pl.ds(j * 128, 128)],
                              cf_hbm.at[pb], semp).wait()


_sc_cp = pltpu.CompilerParams()
if "needs_layout_passes" in pltpu.CompilerParams.__dataclass_fields__:
    _sc_cp = dataclasses.replace(_sc_cp, needs_layout_passes=False)

_sc_prep = pl.kernel(
    _sc_prep_body,
    mesh=plsc.VectorSubcoreMesh(core_axis_name="c", subcore_axis_name="s"),
    compiler_params=_sc_cp,
    out_type=[
        jax.ShapeDtypeStruct((2 * CAP,), jnp.int32),
        jax.ShapeDtypeStruct((2 * CAP,), jnp.float32),
        jax.ShapeDtypeStruct((2 * (NBUCK + 1),), jnp.int32),
    ],
    scratch_types=[
        pltpu.VMEM((PER,), jnp.int32),
        pltpu.VMEM((PER,), jnp.int32),
        pltpu.VMEM((NHIST,), jnp.float32),
        pltpu.VMEM((NHIST,), jnp.float32),
        pltpu.VMEM((16 * RSL,), jnp.float32),
        pltpu.VMEM((RSL,), jnp.float32),
        pltpu.VMEM((1280 + 16,), jnp.float32),
        pltpu.VMEM((96,), jnp.float32),
        pltpu.VMEM((96,), jnp.int32),
        pltpu.VMEM((96,), jnp.int32),
        pltpu.VMEM((NPOS,), jnp.float32),
        pltpu.VMEM((NPOS,), jnp.int32),
        pltpu.VMEM((NPOS,), jnp.int32),
        pltpu.VMEM((128,), jnp.int32),
        pltpu.VMEM((128,), jnp.int32),
        pltpu.VMEM((128,), jnp.int32),
        pltpu.VMEM((128,), jnp.int32),
        pltpu.VMEM((PINIT,), jnp.int32),
        pltpu.VMEM((PINIT,), jnp.float32),
        pltpu.VMEM_SHARED((16 * NHIST + 112,), jnp.float32),
        pltpu.VMEM_SHARED((16 * NHIST + 112,), jnp.float32),
        pltpu.VMEM_SHARED((NHIST,), jnp.float32),
        pltpu.VMEM_SHARED((NHIST,), jnp.float32),
        pltpu.VMEM_SHARED((1280,), jnp.float32),
        pltpu.SMEM((96,), jnp.int32),
        pltpu.SemaphoreType.DMA,
        pltpu.SemaphoreType.DMA,
    ],
)


def _prep_jax(ei):
    src, dst = ei[0], ei[1]
    E = src.shape[0]
    ones = jnp.ones((E,), jnp.float32)
    deg_out = jax.ops.segment_sum(ones, src, num_segments=N)
    deg_in = jax.ops.segment_sum(ones, dst, num_segments=N)
    ns = jax.lax.rsqrt(jnp.maximum(deg_out, 1.0))
    nd = jax.lax.rsqrt(jnp.maximum(deg_in, 1.0))
    coef = ns[src] * nd[dst]
    bucket = dst // BROWS
    order = jnp.argsort(bucket, stable=True)
    srcs = src[order]
    dsts = dst[order]
    coefs = coef[order]
    cnt = jax.ops.segment_sum(jnp.ones((E,), jnp.int32), bucket,
                              num_segments=NBUCK)
    padded = ((cnt + CHUNK - 1) // CHUNK) * CHUNK
    off = jnp.concatenate(
        [jnp.zeros((1,), jnp.int32), jnp.cumsum(padded, dtype=jnp.int32)])
    start = jnp.cumsum(cnt) - cnt
    bsort = bucket[order]
    rank = jnp.arange(E, dtype=jnp.int32) - start[bsort]
    pos = off[bsort] + rank
    packed = (jnp.arange(CAP, dtype=jnp.int32) % N) * BROWS
    packed = packed.at[pos].set(srcs * BROWS + (dsts % BROWS))
    coefa = jnp.zeros((CAP,), jnp.float32).at[pos].set(coefs)
    return packed, coefa, off


def kernel(x, edge_index_rel0, edge_index_rel1, W00, b00, W01, b01, W10, b10, W11, b11):
    b00 = b00.reshape(1, D)
    b01 = b01.reshape(1, D)
    b10 = b10.reshape(1, D)
    b11 = b11.reshape(1, D)
    zpad = jnp.zeros((80,), jnp.int32)
    s_all = jnp.concatenate([edge_index_rel0[0], edge_index_rel1[0], zpad])
    d_all = jnp.concatenate([edge_index_rel0[1], edge_index_rel1[1], zpad])
    packed, coef, off = _sc_prep(s_all, d_all)
    t0, t1 = _tc_mm2(x, W00, W01)
    p_a = _sc_prop(t0, t1, packed, coef, off).reshape(2 * N, D)
    t2, t3 = _tc_mid(p_a, p_a, b00, b01, W10, W11)
    p_b = _sc_prop(t2, t3, packed, coef, off).reshape(2 * N, D)
    return _tc_fin(p_b, p_b, b10, b11, x)


# speedup vs baseline: 1.7306x; 1.0000x over previous
import dataclasses

import jax
import jax.numpy as jnp
from jax import lax
from jax.experimental import pallas as pl
from jax.experimental.pallas import tpu as pltpu
from jax.experimental.pallas import tpu_sc as plsc

N = 10000
D = 512
BLK = 400

BROWS = 128
NBUCK = (N + BROWS - 1) // BROWS
NITEM = 2 * NBUCK
CHUNK = 48
E = 78000
DATA_CAP = 81696
CAP = DATA_CAP + 256
PER = 4880
NGRP = PER // 16
NHIST = 10128
SENT_NODE = 10008
SENT_DST = 10112
RSL = 640
PINIT = 5136
NPOS = 4992


def _mm2_body(x_ref, w0_ref, w1_ref, t0_ref, t1_ref):
    xb = x_ref[...]
    t0_ref[...] = jnp.dot(xb, w0_ref[...], preferred_element_type=jnp.float32)
    t1_ref[...] = jnp.dot(xb, w1_ref[...], preferred_element_type=jnp.float32)


def _tc_mm2(x, w0, w1):
    grid = (N // BLK,)
    return pl.pallas_call(
        _mm2_body,
        grid=grid,
        in_specs=[
            pl.BlockSpec((BLK, D), lambda i: (i, 0)),
            pl.BlockSpec((D, D), lambda i: (0, 0)),
            pl.BlockSpec((D, D), lambda i: (0, 0)),
        ],
        out_specs=[
            pl.BlockSpec((BLK, D), lambda i: (i, 0)),
            pl.BlockSpec((BLK, D), lambda i: (i, 0)),
        ],
        out_shape=[
            jax.ShapeDtypeStruct((N, D), jnp.float32),
            jax.ShapeDtypeStruct((N, D), jnp.float32),
        ],
    )(x, w0, w1)


def _mid_body(p0_ref, p1_ref, b0_ref, b1_ref, w0_ref, w1_ref, t0_ref, t1_ref):
    h = jax.nn.relu(p0_ref[...] + p1_ref[...] + b0_ref[...] + b1_ref[...])
    t0_ref[...] = jnp.dot(h, w0_ref[...], preferred_element_type=jnp.float32)
    t1_ref[...] = jnp.dot(h, w1_ref[...], preferred_element_type=jnp.float32)


def _tc_mid(p0, p1, b0, b1, w0, w1):
    grid = (N // BLK,)
    return pl.pallas_call(
        _mid_body,
        grid=grid,
        in_specs=[
            pl.BlockSpec((BLK, D), lambda i: (i, 0)),
            pl.BlockSpec((BLK, D), lambda i: (i + N // BLK, 0)),
            pl.BlockSpec((1, D), lambda i: (0, 0)),
            pl.BlockSpec((1, D), lambda i: (0, 0)),
            pl.BlockSpec((D, D), lambda i: (0, 0)),
            pl.BlockSpec((D, D), lambda i: (0, 0)),
        ],
        out_specs=[
            pl.BlockSpec((BLK, D), lambda i: (i, 0)),
            pl.BlockSpec((BLK, D), lambda i: (i, 0)),
        ],
        out_shape=[
            jax.ShapeDtypeStruct((N, D), jnp.float32),
            jax.ShapeDtypeStruct((N, D), jnp.float32),
        ],
    )(p0, p1, b0, b1, w0, w1)


def _fin_body(p0_ref, p1_ref, b0_ref, b1_ref, x_ref, o_ref):
    h = jax.nn.relu(p0_ref[...] + p1_ref[...] + b0_ref[...] + b1_ref[...])
    o_ref[...] = jax.nn.relu(h + x_ref[...])


def _tc_fin(p0, p1, b0, b1, x):
    grid = (N // BLK,)
    return pl.pallas_call(
        _fin_body,
        grid=grid,
        in_specs=[
            pl.BlockSpec((BLK, D), lambda i: (i, 0)),
            pl.BlockSpec((BLK, D), lambda i: (i + N // BLK, 0)),
            pl.BlockSpec((1, D), lambda i: (0, 0)),
            pl.BlockSpec((1, D), lambda i: (0, 0)),
            pl.BlockSpec((BLK, D), lambda i: (i, 0)),
        ],
        out_specs=pl.BlockSpec((BLK, D), lambda i: (i, 0)),
        out_shape=jax.ShapeDtypeStruct((N, D), jnp.float32),
    )(p0, p1, b0, b1, x)


def _sc_prop_body(t0_hbm, t1_hbm, pk_hbm, cf_hbm, off_hbm, p_hbm,
                  acc, rows0, rows1, pk0b, pk1b, cf0b, cf1b,
                  idx0, idx1, dstl0, dstl1, offs_v, semA0, semA1, semM):
    wid = lax.axis_index("c") * 16 + lax.axis_index("s")
    pltpu.sync_copy(off_hbm, offs_v.at[pl.ds(0, 2 * (NBUCK + 1))])
    nlast = N - (NBUCK - 1) * BROWS

    @pl.loop(0, (NITEM + 31) // 32)
    def _(k):
        item = wid + 32 * k

        @pl.when(item < NITEM)
        def _():
            r = item // NBUCK
            b = item - r * NBUCK
            zero = jnp.zeros((16,), jnp.float32)

            @plsc.parallel_loop(0, BROWS * D // 256, 1, unroll=2)
            def _(rr):
                for j in range(16):
                    acc[pl.ds(rr * 256 + j * 16, 16)] = zero

            ovec = offs_v[pl.ds(r * (NBUCK + 1) + b, 16)]
            lo = pl.multiple_of(ovec[0], CHUNK)
            hi = ovec[1]
            nch = (hi - lo) // CHUNK
            rbase = pl.multiple_of(r * CAP + lo, 8)

            def meta_start(ci, pkb, cfb):
                pltpu.async_copy(
                    pk_hbm.at[pl.ds(rbase + ci * CHUNK, CHUNK)], pkb, semM)
                pltpu.async_copy(
                    cf_hbm.at[pl.ds(rbase + ci * CHUNK, CHUNK)],
                    cfb.at[pl.ds(0, CHUNK)], semM)

            def meta_wait(ci, pkb, cfb):
                pltpu.make_async_copy(
                    pk_hbm.at[pl.ds(rbase + ci * CHUNK, CHUNK)], pkb,
                    semM).wait()
                pltpu.make_async_copy(
                    cf_hbm.at[pl.ds(rbase + ci * CHUNK, CHUNK)],
                    cfb.at[pl.ds(0, CHUNK)], semM).wait()

            def build(pkb, idxb, dstlb):
                for j in range(CHUNK // 16):
                    v = pkb[pl.ds(j * 16, 16)]
                    idxb[pl.ds(j * 16, 16)] = lax.shift_right_logical(v, 7)
                    dstlb[pl.ds(j * 16, 16)] = lax.bitwise_and(v, BROWS - 1)

            def gather_start(idxb, rowsb, semX):
                @pl.when(r == 0)
                def _():
                    pltpu.async_copy(t0_hbm.at[idxb], rowsb, semX)

                @pl.when(r == 1)
                def _():
                    pltpu.async_copy(t1_hbm.at[idxb], rowsb, semX)

            def gather_wait(idxb, rowsb, semX):
                @pl.when(r == 0)
                def _():
                    pltpu.make_async_copy(t0_hbm.at[idxb], rowsb, semX).wait()

                @pl.when(r == 1)
                def _():
                    pltpu.make_async_copy(t1_hbm.at[idxb], rowsb, semX).wait()

            def compute(rowsb, cfb, dstlb):
                @plsc.parallel_loop(0, CHUNK, 1, unroll=8)
                def _(i):
                    cs = cfb[pl.ds(i, 16)][0]
                    dbase = dstlb[pl.ds(i, 16)][0] * D
                    for j in range(D // 16):
                        v = rowsb[pl.ds(i, 1), pl.ds(j * 16, 16)]
                        plsc.addupdate(
                            acc.at[pl.ds(dbase + j * 16, 16)],
                            (cs * v).reshape(16))

            @pl.when(nch > 0)
            def _():
                meta_start(0, pk0b, cf0b)
                meta_wait(0, pk0b, cf0b)
                build(pk0b, idx0, dstl0)
                gather_start(idx0, rows0, semA0)

                @pl.when(nch > 1)
                def _():
                    meta_start(1, pk1b, cf1b)

            def stage_body(ci, pkb_n, cfb_n, idxb_n, dstlb_n, rowsb_n, semN,
                           pkb_c, cfb_c, idxb_c, dstlb_c, rowsb_c, semC):
                @pl.when(ci + 1 < nch)
                def _():
                    meta_wait(ci + 1, pkb_n, cfb_n)
                    build(pkb_n, idxb_n, dstlb_n)
                    gather_start(idxb_n, rowsb_n, semN)

                gather_wait(idxb_c, rowsb_c, semC)
                compute(rowsb_c, cfb_c, dstlb_c)

                @pl.when(ci + 2 < nch)
                def _():
                    meta_start(ci + 2, pkb_c, cfb_c)

            @pl.loop(0, nch)
            def _(ci):
                @pl.when(lax.rem(ci, 2) == 0)
                def _():
                    stage_body(ci, pk1b, cf1b, idx1, dstl1, rows1, semA1,
                               pk0b, cf0b, idx0, dstl0, rows0, semA0)

                @pl.when(lax.rem(ci, 2) == 1)
                def _():
                    stage_body(ci, pk0b, cf0b, idx0, dstl0, rows0, semA0,
                               pk1b, cf1b, idx1, dstl1, rows1, semA1)

            row0 = (r * N + b * BROWS) * D

            @pl.when(b < NBUCK - 1)
            def _():
                pltpu.sync_copy(acc, p_hbm.at[pl.ds(row0, BROWS * D)])

            @pl.when(b == NBUCK - 1)
            def _():
                pltpu.sync_copy(acc.at[pl.ds(0, nlast * D)],
                                p_hbm.at[pl.ds(row0, nlast * D)])


_sc_prop = pl.kernel(
    _sc_prop_body,
    mesh=plsc.VectorSubcoreMesh(core_axis_name="c", subcore_axis_name="s"),
    out_type=jax.ShapeDtypeStruct((2 * N * D,), jnp.float32),
    scratch_types=[
        pltpu.VMEM((BROWS * D,), jnp.float32),
        pltpu.VMEM((CHUNK, D), jnp.float32),
        pltpu.VMEM((CHUNK, D), jnp.float32),
        pltpu.VMEM((CHUNK,), jnp.int32),
        pltpu.VMEM((CHUNK,), jnp.int32),
        pltpu.VMEM((CHUNK + 16,), jnp.float32),
        pltpu.VMEM((CHUNK + 16,), jnp.float32),
        pltpu.VMEM((CHUNK,), jnp.int32),
        pltpu.VMEM((CHUNK,), jnp.int32),
        pltpu.VMEM((CHUNK + 16,), jnp.int32),
        pltpu.VMEM((CHUNK + 16,), jnp.int32),
        pltpu.VMEM((2 * (NBUCK + 1) + 16,), jnp.int32),
        pltpu.SemaphoreType.DMA,
        pltpu.SemaphoreType.DMA,
        pltpu.SemaphoreType.DMA,
    ],
)


def _quake_rsqrt(v):
    i = plsc.bitcast(v, jnp.int32)
    i = 0x5F3759DF - lax.shift_right_logical(i, 1)
    y = plsc.bitcast(i, jnp.float32)
    for _ in range(3):
        y = y * (1.5 - 0.5 * v * y * y)
    return y


ELAST = E - 15 * PER
ILAST = CAP - 15 * PINIT
RLAST = NHIST - 15 * RSL


def _sc_prep_body(s_hbm, d_hbm, pk_hbm, cf_hbm, off_hbm,
                  sv, dv, hs, hd, rbuf, cbuf, gv, lc, offv, msv,
                  cev, pkv, posv, pb0, pb1, pb2, pb3, ibuf, zbuf,
                  stage_s, stage_d, comb_s, comb_d, grid_spm,
                  cnt_s, sem, semp):
    c = lax.axis_index("c")
    t = lax.axis_index("s")
    ones16 = jnp.full((16,), 1.0, jnp.float32)
    z16f = jnp.zeros((16,), jnp.float32)
    iota16 = lax.iota(jnp.int32, 16)

    pltpu.sync_copy(s_hbm.at[pl.ds(c * E + t * PER, PER)], sv)
    pltpu.sync_copy(d_hbm.at[pl.ds(c * E + t * PER, PER)], dv)

    @pl.when(t == 15)
    def _():
        for j in range((PER - ELAST) // 16):
            sv[pl.ds(ELAST + 16 * j, 16)] = jnp.full((16,), SENT_NODE, jnp.int32)
            dv[pl.ds(ELAST + 16 * j, 16)] = jnp.full((16,), SENT_DST, jnp.int32)

    @plsc.parallel_loop(0, NHIST // 16, 1, unroll=4)
    def _(k):
        hs[pl.ds(k * 16, 16)] = z16f
        hd[pl.ds(k * 16, 16)] = z16f

    for k in range(6):
        lc[pl.ds(k * 16, 16)] = z16f

    @plsc.parallel_loop(0, NGRP, 1, unroll=4)
    def _(g):
        svv = sv[pl.ds(g * 16, 16)]
        dvv = dv[pl.ds(g * 16, 16)]
        plsc.addupdate_scatter(hs, [svv], ones16)
        plsc.addupdate_scatter(hd, [dvv], ones16)
        plsc.addupdate_scatter(lc, [lax.shift_right_logical(dvv, 7)], ones16)

    pltpu.sync_copy(hs, stage_s.at[pl.ds(t * NHIST, NHIST)])
    pltpu.sync_copy(hd, stage_d.at[pl.ds(t * NHIST, NHIST)])
    pltpu.sync_copy(lc.at[pl.ds(0, 80)], grid_spm.at[pl.ds(t * 80, 80)])
    plsc.subcore_barrier()

    def _reduce(stage, comb):
        for tt in range(16):
            pltpu.async_copy(stage.at[pl.ds(tt * NHIST + t * RSL, RSL)],
                             rbuf.at[pl.ds(tt * RSL, RSL)], sem)
        for tt in range(16):
            pltpu.make_async_copy(stage.at[pl.ds(tt * NHIST + t * RSL, RSL)],
                                  rbuf.at[pl.ds(tt * RSL, RSL)], sem).wait()

        @plsc.parallel_loop(0, RSL // 16, 1, unroll=2)
        def _(k):
            s = rbuf[pl.ds(k * 16, 16)]
            for tt in range(1, 16):
                s = s + rbuf[pl.ds(tt * RSL + k * 16, 16)]
            cbuf[pl.ds(k * 16, 16)] = s

        @pl.when(t < 15)
        def _():
            pltpu.sync_copy(cbuf, comb.at[pl.ds(t * RSL, RSL)])

        @pl.when(t == 15)
        def _():
            pltpu.sync_copy(cbuf.at[pl.ds(0, RLAST)],
                            comb.at[pl.ds(t * RSL, RLAST)])

    _reduce(stage_s, comb_s)
    _reduce(stage_d, comb_d)
    plsc.subcore_barrier()

    pltpu.sync_copy(comb_s, hs)
    pltpu.sync_copy(comb_d, hd)

    @plsc.parallel_loop(0, NHIST // 16, 1, unroll=4)
    def _(k):
        hs[pl.ds(k * 16, 16)] = _quake_rsqrt(
            jnp.maximum(hs[pl.ds(k * 16, 16)], 1.0))
        hd[pl.ds(k * 16, 16)] = _quake_rsqrt(
            jnp.maximum(hd[pl.ds(k * 16, 16)], 1.0))

    @plsc.parallel_loop(0, NGRP, 1, unroll=4)
    def _(g):
        svv = sv[pl.ds(g * 16, 16)]
        dvv = dv[pl.ds(g * 16, 16)]
        cev[pl.ds(g * 16, 16)] = (plsc.load_gather(hs, [svv]) *
                                  plsc.load_gather(hd, [dvv]))

    pltpu.sync_copy(grid_spm, gv.at[pl.ds(0, 1280)])
    carry = jnp.int32(0)
    for k in range(5):
        s = gv[pl.ds(k * 16, 16)]
        for tt in range(1, 16):
            s = s + gv[pl.ds(tt * 80 + k * 16, 16)]
        cnt_i = s.astype(jnp.int32)
        pad = ((cnt_i + (CHUNK - 1)) // CHUNK) * CHUNK
        if k == 4:
            pad = jnp.where(iota16 == 15, 0, pad)
        incl = plsc.cumsum(pad) + carry
        offv[pl.ds(k * 16, 16)] = incl - pad
        carry = incl[15]

    @pl.when(t == 0)
    def _():
        pltpu.sync_copy(offv.at[pl.ds(0, 80)], off_hbm.at[pl.ds(c * 80, 80)])

    for k in range(5):
        msv[pl.ds(k * 16, 16)] = offv[pl.ds(k * 16, 16)] + c * CAP

    @pl.loop(0, t)
    def _(tt):
        for k in range(5):
            msv[pl.ds(k * 16, 16)] = (msv[pl.ds(k * 16, 16)] +
                                      gv[pl.ds(tt * 80 + k * 16, 16)].astype(jnp.int32))

    for k in range(5):
        v = msv[pl.ds(k * 16, 16)]
        for lane in range(16):
            cnt_s[k * 16 + lane] = v[lane]

    @plsc.parallel_loop(0, PINIT // 16, 1, unroll=4)
    def _(j):
        g = t * PINIT + j * 16 + iota16
        ibuf[pl.ds(j * 16, 16)] = lax.rem(g, N) * BROWS
        zbuf[pl.ds(j * 16, 16)] = z16f

    @pl.when(t < 15)
    def _():
        pltpu.sync_copy(ibuf, pk_hbm.at[pl.ds(c * CAP + t * PINIT, PINIT)])
        pltpu.sync_copy(zbuf, cf_hbm.at[pl.ds(c * CAP + t * PINIT, PINIT)])

    @pl.when(t == 15)
    def _():
        pltpu.sync_copy(ibuf.at[pl.ds(0, ILAST)],
                        pk_hbm.at[pl.ds(c * CAP + 15 * PINIT, ILAST)])
        pltpu.sync_copy(zbuf.at[pl.ds(0, ILAST)],
                        cf_hbm.at[pl.ds(c * CAP + 15 * PINIT, ILAST)])

    plsc.subcore_barrier()

    @plsc.parallel_loop(0, NPOS // 16, 1, unroll=4)
    def _(j):
        posv[pl.ds(j * 16, 16)] = (c * CAP + DATA_CAP +
                                   lax.rem(j * 16 + iota16, 256))

    @pl.loop(0, NGRP)
    def _(g):
        dvv = dv[pl.ds(g * 16, 16)]
        bv = lax.shift_right_logical(dvv, 7)
        pv = jnp.zeros((16,), jnp.int32)
        for lane in range(16):
            b = bv[lane]
            p = cnt_s[b]
            cnt_s[b] = p + 1
            pv = jnp.where(iota16 == lane, p, pv)
        posv[pl.ds(g * 16, 16)] = pv
        pkv[pl.ds(g * 16, 16)] = (sv[pl.ds(g * 16, 16)] * BROWS +
                                  lax.bitwise_and(dvv, BROWS - 1))

    nsc = NPOS // 128
    pbs = [pb0, pb1, pb2, pb3]
    nd = len(pbs)
    for j in range(nsc):
        pb = pbs[j % nd]
        if j >= nd:
            pltpu.make_async_copy(pkv.at[pl.ds((j - nd) * 128, 128)],
                                  pk_hbm.at[pb], semp).wait()
            pltpu.make_async_copy(cev.at[pl.ds((j - nd) * 128, 128)],
                                  cf_hbm.at[pb], semp).wait()
        for k in range(8):
            pb[pl.ds(k * 16, 16)] = posv[pl.ds(j * 128 + k * 16, 16)]
        pltpu.async_copy(pkv.at[pl.ds(j * 128, 128)], pk_hbm.at[pb], semp)
        pltpu.async_copy(cev.at[pl.ds(j * 128, 128)], cf_hbm.at[pb], semp)
    for j in range(nsc - nd, nsc):
        pb = pbs[j % nd]
        pltpu.make_async_copy(pkv.at[pl.ds(j * 128, 128)],
                              pk_hbm.at[pb], semp).wait()
        pltpu.make_async_copy(cev.at[pl.ds(j * 128, 128)],
                              cf_hbm.at[pb], semp).wait()


_sc_cp = pltpu.CompilerParams()
if "needs_layout_passes" in pltpu.CompilerParams.__dataclass_fields__:
    _sc_cp = dataclasses.replace(_sc_cp, needs_layout_passes=False)

_sc_prep = pl.kernel(
    _sc_prep_body,
    mesh=plsc.VectorSubcoreMesh(core_axis_name="c", subcore_axis_name="s"),
    compiler_params=_sc_cp,
    out_type=[
        jax.ShapeDtypeStruct((2 * CAP,), jnp.int32),
        jax.ShapeDtypeStruct((2 * CAP,), jnp.float32),
        jax.ShapeDtypeStruct((2 * (NBUCK + 1),), jnp.int32),
    ],
    scratch_types=[
        pltpu.VMEM((PER,), jnp.int32),
        pltpu.VMEM((PER,), jnp.int32),
        pltpu.VMEM((NHIST,), jnp.float32),
        pltpu.VMEM((NHIST,), jnp.float32),
        pltpu.VMEM((16 * RSL,), jnp.float32),
        pltpu.VMEM((RSL,), jnp.float32),
        pltpu.VMEM((1280 + 16,), jnp.float32),
        pltpu.VMEM((96,), jnp.float32),
        pltpu.VMEM((96,), jnp.int32),
        pltpu.VMEM((96,), jnp.int32),
        pltpu.VMEM((NPOS,), jnp.float32),
        pltpu.VMEM((NPOS,), jnp.int32),
        pltpu.VMEM((NPOS,), jnp.int32),
        pltpu.VMEM((128,), jnp.int32),
        pltpu.VMEM((128,), jnp.int32),
        pltpu.VMEM((128,), jnp.int32),
        pltpu.VMEM((128,), jnp.int32),
        pltpu.VMEM((PINIT,), jnp.int32),
        pltpu.VMEM((PINIT,), jnp.float32),
        pltpu.VMEM_SHARED((16 * NHIST + 112,), jnp.float32),
        pltpu.VMEM_SHARED((16 * NHIST + 112,), jnp.float32),
        pltpu.VMEM_SHARED((NHIST,), jnp.float32),
        pltpu.VMEM_SHARED((NHIST,), jnp.float32),
        pltpu.VMEM_SHARED((1280,), jnp.float32),
        pltpu.SMEM((96,), jnp.int32),
        pltpu.SemaphoreType.DMA,
        pltpu.SemaphoreType.DMA,
    ],
)


def kernel(x, edge_index_rel0, edge_index_rel1, W00, b00, W01, b01, W10, b10, W11, b11):
    b00 = b00.reshape(1, D)
    b01 = b01.reshape(1, D)
    b10 = b10.reshape(1, D)
    b11 = b11.reshape(1, D)
    zpad = jnp.zeros((80,), jnp.int32)
    s_all = jnp.concatenate([edge_index_rel0[0], edge_index_rel1[0], zpad])
    d_all = jnp.concatenate([edge_index_rel0[1], edge_index_rel1[1], zpad])
    packed, coef, off = _sc_prep(s_all, d_all)
    t0, t1 = _tc_mm2(x, W00, W01)
    p_a = _sc_prop(t0, t1, packed, coef, off).reshape(2 * N, D)
    t2, t3 = _tc_mid(p_a, p_a, b00, b01, W10, W11)
    p_b = _sc_prop(t2, t3, packed, coef, off).reshape(2 * N, D)
    return _tc_fin(p_b, p_b, b10, b11, x)
